# baseline (speedup 1.0000x reference)
import functools
import jax
import jax.numpy as jnp
from jax import lax
from jax.experimental import pallas as pl
from jax.experimental.pallas import tpu as pltpu
from jax.experimental.pallas import tpu_sc as plsc

N = 10000
E = 320000
D = 128
EP = 331776
TE = 4096
GRID_E = EP // TE
NTILE = 1000
GRID_N = N // NTILE

NW = 32
PER_W = EP // NW
CH = 128
NCH = PER_W // CH
PER_T = EP // 16
NCH_S = PER_T // CH
N_PAD = 10240
NROW = N_PAD // 16

_f32 = jnp.float32


def _node_body(x_ref, Win_ref, bin_ref, Wlin_ref, Wd_ref, Ws_ref, aW1_ref,
               x1_ref, xv_ref, Pp_ref, Qp_ref):
    x1 = jnp.maximum(x_ref[...] @ Win_ref[...] + bin_ref[...], 0.0)
    x1_ref[...] = x1
    xv_ref[...] = (x1 @ Wlin_ref[...]).astype(jnp.bfloat16)
    Pp_ref[...] = (x1 @ (Wd_ref[...] @ aW1_ref[...])).astype(jnp.bfloat16)
    Qp_ref[...] = (x1 @ (Ws_ref[...] @ aW1_ref[...])).astype(jnp.bfloat16)


def _wspec():
    return pl.BlockSpec((D, D), lambda i: (0, 0))


def _rspec():
    return pl.BlockSpec((1, D), lambda i: (0, 0))


def _node_call(x, W_in, b_in, W_lin, W_d, W_s, aW1):
    nspec = pl.BlockSpec((NTILE, D), lambda i: (i, 0))
    outs = [jax.ShapeDtypeStruct((N, D), _f32)] + [jax.ShapeDtypeStruct((N, D), jnp.bfloat16)] * 3
    return pl.pallas_call(
        _node_body,
        grid=(GRID_N,),
        in_specs=[nspec, _wspec(), _rspec(), _wspec(), _wspec(), _wspec(), _wspec()],
        out_specs=[nspec] * 4,
        out_shape=outs,
    )(x, W_in, b_in, W_lin, W_d, W_s, aW1)


def _gather_pos_call(dstA, srcA, pos16):
    mesh = plsc.VectorSubcoreMesh(core_axis_name="c", subcore_axis_name="s")

    @functools.partial(
        pl.kernel,
        mesh=mesh,
        out_type=[
            jax.ShapeDtypeStruct((EP, 16), _f32),
            jax.ShapeDtypeStruct((EP, 16), _f32),
        ],
        scratch_types=[
            pltpu.VMEM((CH,), jnp.int32),
            pltpu.VMEM((CH,), jnp.int32),
            pltpu.VMEM((CH, 16), _f32),
            pltpu.VMEM((CH, 16), _f32),
            pltpu.SemaphoreType.DMA,
        ],
        compiler_params=pltpu.CompilerParams(use_tc_tiling_on_sc=False),
    )
    def k(dst_hbm, src_hbm, pos_hbm, posd_hbm, poss_hbm,
          idxd, idxs, bpd, bps, sem):
        wid = lax.axis_index("s") * 2 + lax.axis_index("c")
        base = wid * PER_W

        def body(ci, carry):
            off = base + ci * CH
            pltpu.sync_copy(dst_hbm.at[pl.ds(off, CH)], idxd)
            pltpu.sync_copy(src_hbm.at[pl.ds(off, CH)], idxs)
            c4 = pltpu.async_copy(pos_hbm.at[idxd], bpd, sem)
            c5 = pltpu.async_copy(pos_hbm.at[idxs], bps, sem)
            c4.wait(); c5.wait()
            pltpu.sync_copy(bpd, posd_hbm.at[pl.ds(off, CH)])
            pltpu.sync_copy(bps, poss_hbm.at[pl.ds(off, CH)])
            return carry

        lax.fori_loop(0, NCH, body, 0)

    return k(dstA, srcA, pos16)


def _gather_main_call(dstA, srcA, Pp, Qp, xv):
    mesh = plsc.VectorSubcoreMesh(core_axis_name="c", subcore_axis_name="s")

    @functools.partial(
        pl.kernel,
        mesh=mesh,
        out_type=[
            jax.ShapeDtypeStruct((EP, D), jnp.bfloat16),
            jax.ShapeDtypeStruct((EP, D), jnp.bfloat16),
            jax.ShapeDtypeStruct((EP, D), jnp.bfloat16),
        ],
        scratch_types=[
            pltpu.VMEM((CH,), jnp.int32),
            pltpu.VMEM((CH,), jnp.int32),
            pltpu.VMEM((CH, D), jnp.bfloat16),
            pltpu.VMEM((CH, D), jnp.bfloat16),
            pltpu.VMEM((CH, D), jnp.bfloat16),
            pltpu.SemaphoreType.DMA,
        ],
        compiler_params=pltpu.CompilerParams(use_tc_tiling_on_sc=False),
    )
    def k(dst_hbm, src_hbm, Pp_hbm, Qp_hbm, xv_hbm,
          Pd_hbm, Qs_hbm, xvs_hbm,
          idxd, idxs, bP, bQ, bX, sem):
        wid = lax.axis_index("s") * 2 + lax.axis_index("c")
        base = wid * PER_W

        def body(ci, carry):
            off = base + ci * CH
            pltpu.sync_copy(dst_hbm.at[pl.ds(off, CH)], idxd)
            pltpu.sync_copy(src_hbm.at[pl.ds(off, CH)], idxs)
            c1 = pltpu.async_copy(Pp_hbm.at[idxd], bP, sem)
            c2 = pltpu.async_copy(Qp_hbm.at[idxs], bQ, sem)
            c3 = pltpu.async_copy(xv_hbm.at[idxs], bX, sem)
            c1.wait(); c2.wait(); c3.wait()
            pltpu.sync_copy(bP, Pd_hbm.at[pl.ds(off, CH)])
            pltpu.sync_copy(bQ, Qs_hbm.at[pl.ds(off, CH)])
            pltpu.sync_copy(bX, xvs_hbm.at[pl.ds(off, CH)])
            return carry

        lax.fori_loop(0, NCH, body, 0)

    return k(dstA, srcA, Pp, Qp, xv)


def _scatter_call(dstA, ex, evd, zeros_nd):
    mesh = plsc.VectorSubcoreMesh(core_axis_name="c", subcore_axis_name="s")

    @functools.partial(
        pl.kernel,
        mesh=mesh,
        out_type=[
            jax.ShapeDtypeStruct((N_PAD, D), _f32),
            jax.ShapeDtypeStruct((N_PAD, D), _f32),
        ],
        scratch_types=[
            pltpu.VMEM((CH,), jnp.int32),
            pltpu.VMEM((CH, D), _f32),
            pltpu.VMEM_SHARED((N_PAD, D), _f32),
        ],
    )
    def k(dst_hbm, ex_hbm, evd_hbm, z_hbm, s_hbm, num_hbm, idx, rows, acc):
        c = lax.axis_index("c")
        sid = lax.axis_index("s")
        pltpu.sync_copy(z_hbm.at[pl.ds(sid * NROW, NROW)],
                        acc.at[pl.ds(sid * NROW, NROW)])
        plsc.subcore_barrier()

        def run(data_hbm):
            def body(ci, carry):
                off = sid * PER_T + ci * CH
                pltpu.sync_copy(dst_hbm.at[pl.ds(off, CH)], idx)
                pltpu.sync_copy(data_hbm.at[pl.ds(off, CH)], rows)
                pltpu.sync_copy(rows, acc.at[idx], add=True)
                return carry
            lax.fori_loop(0, NCH_S, body, 0)

        @pl.when(c == 0)
        def _():
            run(ex_hbm)

        @pl.when(c == 1)
        def _():
            run(evd_hbm)

        plsc.subcore_barrier()

        @pl.when(c == 0)
        def _():
            pltpu.sync_copy(acc.at[pl.ds(sid * NROW, NROW)],
                            s_hbm.at[pl.ds(sid * NROW, NROW)])

        @pl.when(c == 1)
        def _():
            pltpu.sync_copy(acc.at[pl.ds(sid * NROW, NROW)],
                            num_hbm.at[pl.ds(sid * NROW, NROW)])

    return k(dstA, ex, evd, zeros_nd)


def _espec(cols, colblk=0):
    return pl.BlockSpec((TE, cols), lambda i, _c=colblk: (i, _c))


def _acc_init(ref):
    @pl.when(pl.program_id(0) == 0)
    def _():
        ref[...] = jnp.zeros_like(ref)


def _t1_body(posd_ref, poss_ref, w_ref, pW1_ref, pb1_ref,
             s_ref, ss_ref, cnt_ref):
    rel = posd_ref[...] - poss_ref[...]
    h1 = rel @ pW1_ref[...] + pb1_ref[...]
    w = w_ref[...]
    wh = h1 * w
    _acc_init(s_ref); _acc_init(ss_ref); _acc_init(cnt_ref)
    s_ref[...] += jnp.sum(wh, axis=0)[None, :]
    ss_ref[...] += jnp.sum(wh * h1, axis=0)[None, :]
    cnt_ref[...] += jnp.full((1, D), jnp.sum(w), _f32)


def _t2_body(posd_ref, poss_ref, w_ref, pW1_ref, pb1_ref, sc1_ref, t1_ref,
             pW2_ref, pb2_ref, s_ref, ss_ref):
    rel = posd_ref[...] - poss_ref[...]
    h1 = rel @ pW1_ref[...] + pb1_ref[...]
    r = jnp.maximum(h1 * sc1_ref[...] + t1_ref[...], 0.0)
    h2 = r @ pW2_ref[...] + pb2_ref[...]
    w = w_ref[...]
    wh = h2 * w
    _acc_init(s_ref); _acc_init(ss_ref)
    s_ref[...] += jnp.sum(wh, axis=0)[None, :]
    ss_ref[...] += jnp.sum(wh * h2, axis=0)[None, :]


def _delta(posd_ref, poss_ref, pW1_ref, pb1_ref, sc1_ref, t1_ref,
           pW2_ref, pb2_ref, sc2_ref, t2_ref):
    rel = posd_ref[...] - poss_ref[...]
    h1 = rel @ pW1_ref[...] + pb1_ref[...]
    r = jnp.maximum(h1 * sc1_ref[...] + t1_ref[...], 0.0)
    h2 = r @ pW2_ref[...] + pb2_ref[...]
    return jnp.maximum(h2 * sc2_ref[...] + t2_ref[...], 0.0)


def _t3_body(posd_ref, poss_ref, Pd_ref, Qs_ref, w_ref,
             pW1_ref, pb1_ref, sc1_ref, t1_ref, pW2_ref, pb2_ref,
             sc2_ref, t2_ref, aW1_ref, ab1_ref,
             a1p_ref, s_ref, ss_ref):
    delta = _delta(posd_ref, poss_ref, pW1_ref, pb1_ref, sc1_ref, t1_ref,
                   pW2_ref, pb2_ref, sc2_ref, t2_ref)
    a1p = (Pd_ref[...].astype(jnp.float32) - Qs_ref[...].astype(jnp.float32)
           + delta @ aW1_ref[...] + ab1_ref[...])
    a1p_ref[...] = a1p
    w = w_ref[...]
    wh = a1p * w
    _acc_init(s_ref); _acc_init(ss_ref)
    s_ref[...] += jnp.sum(wh, axis=0)[None, :]
    ss_ref[...] += jnp.sum(wh * a1p, axis=0)[None, :]


def _t4_body(a1p_ref, w_ref, sc3_ref, t3_ref, aW2_ref, ab2_ref,
             a2p_ref, s_ref, ss_ref, mx_ref, mn_ref):
    ra = jnp.maximum(a1p_ref[...] * sc3_ref[...] + t3_ref[...], 0.0)
    a2p = ra @ aW2_ref[...] + ab2_ref[...]
    a2p_ref[...] = a2p
    w = w_ref[...]
    wh = a2p * w
    _acc_init(s_ref); _acc_init(ss_ref)
    s_ref[...] += jnp.sum(wh, axis=0)[None, :]
    ss_ref[...] += jnp.sum(wh * a2p, axis=0)[None, :]
    pmx = jnp.max(a2p, axis=0)[None, :]
    pmn = jnp.min(a2p, axis=0)[None, :]

    @pl.when(pl.program_id(0) == 0)
    def _():
        mx_ref[...] = pmx
        mn_ref[...] = pmn

    @pl.when(pl.program_id(0) > 0)
    def _():
        mx_ref[...] = jnp.maximum(mx_ref[...], pmx)
        mn_ref[...] = jnp.minimum(mn_ref[...], pmn)


def _t5_body(a2p_ref, posd_ref, poss_ref, xvs_ref, w_ref,
             pW1_ref, pb1_ref, sc1_ref, t1_ref, pW2_ref, pb2_ref,
             sc2_ref, t2_ref, sc4_ref, t4_ref, M_ref,
             ex_ref, evd_ref):
    delta = _delta(posd_ref, poss_ref, pW1_ref, pb1_ref, sc1_ref, t1_ref,
                   pW2_ref, pb2_ref, sc2_ref, t2_ref)
    af = jnp.maximum(a2p_ref[...] * sc4_ref[...] + t4_ref[...], 0.0)
    ex = w_ref[...] * jnp.exp(af - M_ref[...])
    ex_ref[...] = ex
    evd_ref[...] = ex * (xvs_ref[...].astype(jnp.float32) + delta)


def _t1_call(posd, poss, w2, pW1p, pb1):
    return pl.pallas_call(
        _t1_body,
        grid=(GRID_E,),
        in_specs=[_espec(16), _espec(16), _espec(1),
                  pl.BlockSpec((16, D), lambda i: (0, 0)), _rspec()],
        out_specs=[_rspec(), _rspec(), _rspec()],
        out_shape=[jax.ShapeDtypeStruct((1, D), _f32)] * 3,
    )(posd, poss, w2, pW1p, pb1)


def _t2_call(posd, poss, w2, pW1p, pb1, sc1, t1, pW2, pb2):
    return pl.pallas_call(
        _t2_body,
        grid=(GRID_E,),
        in_specs=[_espec(16), _espec(16), _espec(1),
                  pl.BlockSpec((16, D), lambda i: (0, 0)), _rspec(),
                  _rspec(), _rspec(), _wspec(), _rspec()],
        out_specs=[_rspec(), _rspec()],
        out_shape=[jax.ShapeDtypeStruct((1, D), _f32)] * 2,
    )(posd, poss, w2, pW1p, pb1, sc1, t1, pW2, pb2)


def _t3_call(posd, poss, Pd, Qs, w2, pW1p, pb1, sc1, t1, pW2, pb2,
             sc2, t2, aW1, ab1):
    return pl.pallas_call(
        _t3_body,
        grid=(GRID_E,),
        in_specs=[_espec(16), _espec(16), _espec(D), _espec(D), _espec(1),
                  pl.BlockSpec((16, D), lambda i: (0, 0)), _rspec(),
                  _rspec(), _rspec(), _wspec(), _rspec(),
                  _rspec(), _rspec(), _wspec(), _rspec()],
        out_specs=[_espec(D), _rspec(), _rspec()],
        out_shape=[jax.ShapeDtypeStruct((EP, D), _f32),
                   jax.ShapeDtypeStruct((1, D), _f32),
                   jax.ShapeDtypeStruct((1, D), _f32)],
    )(posd, poss, Pd, Qs, w2, pW1p, pb1, sc1, t1, pW2, pb2, sc2, t2, aW1, ab1)


def _t4_call(a1p, w2, sc3, t3, aW2, ab2):
    return pl.pallas_call(
        _t4_body,
        grid=(GRID_E,),
        in_specs=[_espec(D), _espec(1), _rspec(), _rspec(), _wspec(), _rspec()],
        out_specs=[_espec(D), _rspec(), _rspec(), _rspec(), _rspec()],
        out_shape=[jax.ShapeDtypeStruct((EP, D), _f32)] +
                  [jax.ShapeDtypeStruct((1, D), _f32)] * 4,
    )(a1p, w2, sc3, t3, aW2, ab2)


def _t5_call(a2p, posd, poss, xvs, w2, pW1p, pb1, sc1, t1, pW2, pb2,
             sc2, t2, sc4, t4, M):
    return pl.pallas_call(
        _t5_body,
        grid=(GRID_E,),
        in_specs=[_espec(D), _espec(16), _espec(16), _espec(D), _espec(1),
                  pl.BlockSpec((16, D), lambda i: (0, 0)), _rspec(),
                  _rspec(), _rspec(), _wspec(), _rspec(),
                  _rspec(), _rspec(), _rspec(), _rspec(),
                  pl.BlockSpec((1, 1), lambda i: (0, 0))],
        out_specs=[_espec(D), _espec(D)],
        out_shape=[jax.ShapeDtypeStruct((EP, D), _f32)] * 2,
    )(a2p, posd, poss, xvs, w2, pW1p, pb1, sc1, t1, pW2, pb2, sc2, t2,
      sc4, t4, M)


def _final_body(s_ref, num_ref, x1_ref, Wup_ref, bup_ref, g_ref, b_ref, o_ref):
    o = jnp.maximum((num_ref[...] / s_ref[...]) @ Wup_ref[...] + bup_ref[...],
                    0.0)
    h = o + x1_ref[...]
    mu = jnp.mean(h, axis=1, keepdims=True)
    var = jnp.mean((h - mu) * (h - mu), axis=1, keepdims=True)
    o_ref[...] = (h - mu) / jnp.sqrt(var + 1e-5) * g_ref[...] + b_ref[...]


def _final_call(s, num, x1, W_up, b_up, ln_g, ln_b):
    nspec = pl.BlockSpec((NTILE, D), lambda i: (i, 0))
    return pl.pallas_call(
        _final_body,
        grid=(GRID_N,),
        in_specs=[nspec, nspec, nspec, _wspec(), _rspec(), _rspec(), _rspec()],
        out_specs=nspec,
        out_shape=jax.ShapeDtypeStruct((N, D), _f32),
    )(s, num, x1, W_up, b_up, ln_g, ln_b)


def _bn_params(s, ss, cnt, g, b):
    mu = s / cnt
    var = ss / cnt - mu * mu
    sc = g[None, :] / jnp.sqrt(var + 1e-5)
    t = b[None, :] - mu * sc
    return sc, t


def kernel(x, pos, edge_index, W_in, b_in, W_lin, W_src, W_dst, pW1, pb1,
           pg1, pB1, pW2, pb2, pg2, pB2, aW1, ab1, ag1, aB1, aW2, ab2,
           ag2, aB2, W_up, b_up, ln_g, ln_b):
    src = edge_index[0]
    dst = edge_index[1]
    loop = jnp.arange(N, dtype=src.dtype)
    pad = EP - (E + N)
    srcA = jnp.concatenate([src, loop, jnp.zeros((pad,), src.dtype)])
    dstA = jnp.concatenate([dst, loop, jnp.zeros((pad,), src.dtype)])
    w2 = jnp.concatenate([(src != dst).astype(_f32), jnp.ones((N,), _f32),
                          jnp.zeros((pad,), _f32)]).reshape(EP, 1)
    pos16 = jnp.pad(pos, ((0, 0), (0, 13)))
    pW1p = jnp.pad(pW1, ((0, 13), (0, 0)))
    b_in2 = b_in.reshape(1, D)
    pb1r = pb1.reshape(1, D)
    pb2r = pb2.reshape(1, D)
    ab1r = ab1.reshape(1, D)
    ab2r = ab2.reshape(1, D)

    posd, poss = _gather_pos_call(dstA, srcA, pos16)

    x1, xv, Pp, Qp = _node_call(x, W_in, b_in2, W_lin, W_dst, W_src, aW1)

    Pd, Qs, xvs = _gather_main_call(dstA, srcA, Pp, Qp, xv)

    s1, ss1, cntv = _t1_call(posd, poss, w2, pW1p, pb1r)
    cnt = cntv[0, 0]
    sc1, t1 = _bn_params(s1, ss1, cnt, pg1, pB1)

    s2, ss2 = _t2_call(posd, poss, w2, pW1p, pb1r, sc1, t1, pW2, pb2r)
    sc2, t2 = _bn_params(s2, ss2, cnt, pg2, pB2)

    a1p, s3, ss3 = _t3_call(posd, poss, Pd, Qs, w2, pW1p, pb1r, sc1, t1,
                            pW2, pb2r, sc2, t2, aW1, ab1r)
    sc3, t3 = _bn_params(s3, ss3, cnt, ag1, aB1)

    a2p, s4, ss4, mxc, mnc = _t4_call(a1p, w2, sc3, t3, aW2, ab2r)
    sc4, t4 = _bn_params(s4, ss4, cnt, ag2, aB2)
    M = jnp.maximum(jnp.maximum(mxc * sc4 + t4, mnc * sc4 + t4), 0.0).max()

    ex, evd = _t5_call(a2p, posd, poss, xvs, w2, pW1p, pb1r, sc1, t1,
                       pW2, pb2r, sc2, t2, sc4, t4, M.reshape(1, 1))

    zeros_nd = jnp.zeros((N_PAD, D), _f32)
    ssum, num = _scatter_call(dstA, ex, evd, zeros_nd)
    ssum = ssum[:N]
    num = num[:N]

    return _final_call(ssum, num, x1, W_up, b_up.reshape(1, D), ln_g.reshape(1, D),
                       ln_b.reshape(1, D))


# speedup vs baseline: 1.3253x; 1.3253x over previous
import functools
import jax
import jax.numpy as jnp
from jax import lax
from jax.experimental import pallas as pl
from jax.experimental.pallas import tpu as pltpu
from jax.experimental.pallas import tpu_sc as plsc

N = 10000
E = 320000
D = 128
EP = 331776
TE = 4096
GRID_E = EP // TE
NTILE = 1000
GRID_N = N // NTILE

NW = 32
PER_W = EP // NW
CH = 128
NCH = PER_W // CH
PER_T = EP // 16
NCH_S = PER_T // CH
N_PAD = 10240
NROW = N_PAD // 16

_f32 = jnp.float32


def _node_body(x_ref, Win_ref, bin_ref, Wlin_ref, Wd_ref, Ws_ref, aW1_ref,
               x1_ref, xv_ref, Pp_ref, Qp_ref):
    x1 = jnp.maximum(x_ref[...] @ Win_ref[...] + bin_ref[...], 0.0)
    x1_ref[...] = x1
    xv_ref[...] = x1 @ Wlin_ref[...]
    Pp_ref[...] = x1 @ (Wd_ref[...] @ aW1_ref[...])
    Qp_ref[...] = x1 @ (Ws_ref[...] @ aW1_ref[...])


def _wspec():
    return pl.BlockSpec((D, D), lambda i: (0, 0))


def _rspec():
    return pl.BlockSpec((1, D), lambda i: (0, 0))


def _node_call(x, W_in, b_in, W_lin, W_d, W_s, aW1):
    nspec = pl.BlockSpec((NTILE, D), lambda i: (i, 0))
    outs = [jax.ShapeDtypeStruct((N, D), _f32)] * 4
    return pl.pallas_call(
        _node_body,
        grid=(GRID_N,),
        in_specs=[nspec, _wspec(), _rspec(), _wspec(), _wspec(), _wspec(), _wspec()],
        out_specs=[nspec] * 4,
        out_shape=outs,
    )(x, W_in, b_in, W_lin, W_d, W_s, aW1)


def _gather_pos_call(dstA, srcA, pos16):
    mesh = plsc.VectorSubcoreMesh(core_axis_name="c", subcore_axis_name="s")

    @functools.partial(
        pl.kernel,
        mesh=mesh,
        out_type=[
            jax.ShapeDtypeStruct((EP, 16), _f32),
            jax.ShapeDtypeStruct((EP, 16), _f32),
        ],
        scratch_types=[
            pltpu.VMEM((CH,), jnp.int32),
            pltpu.VMEM((CH,), jnp.int32),
            pltpu.VMEM((CH, 16), _f32),
            pltpu.VMEM((CH, 16), _f32),
            pltpu.SemaphoreType.DMA,
        ],
        compiler_params=pltpu.CompilerParams(use_tc_tiling_on_sc=False),
    )
    def k(dst_hbm, src_hbm, pos_hbm, posd_hbm, poss_hbm,
          idxd, idxs, bpd, bps, sem):
        wid = lax.axis_index("s") * 2 + lax.axis_index("c")
        base = wid * PER_W

        def body(ci, carry):
            off = base + ci * CH
            pltpu.sync_copy(dst_hbm.at[pl.ds(off, CH)], idxd)
            pltpu.sync_copy(src_hbm.at[pl.ds(off, CH)], idxs)
            c4 = pltpu.async_copy(pos_hbm.at[idxd], bpd, sem)
            c5 = pltpu.async_copy(pos_hbm.at[idxs], bps, sem)
            c4.wait(); c5.wait()
            pltpu.sync_copy(bpd, posd_hbm.at[pl.ds(off, CH)])
            pltpu.sync_copy(bps, poss_hbm.at[pl.ds(off, CH)])
            return carry

        lax.fori_loop(0, NCH, body, 0)

    return k(dstA, srcA, pos16)


def _gather_main_call(dstA, srcA, Pp, Qp, xv):
    mesh = plsc.VectorSubcoreMesh(core_axis_name="c", subcore_axis_name="s")

    @functools.partial(
        pl.kernel,
        mesh=mesh,
        out_type=[
            jax.ShapeDtypeStruct((EP, D), _f32),
            jax.ShapeDtypeStruct((EP, D), _f32),
            jax.ShapeDtypeStruct((EP, D), _f32),
        ],
        scratch_types=[
            pltpu.VMEM((CH,), jnp.int32),
            pltpu.VMEM((CH,), jnp.int32),
            pltpu.VMEM((CH, D), _f32),
            pltpu.VMEM((CH, D), _f32),
            pltpu.VMEM((CH, D), _f32),
            pltpu.SemaphoreType.DMA,
        ],
        compiler_params=pltpu.CompilerParams(use_tc_tiling_on_sc=False),
    )
    def k(dst_hbm, src_hbm, Pp_hbm, Qp_hbm, xv_hbm,
          Pd_hbm, Qs_hbm, xvs_hbm,
          idxd, idxs, bP, bQ, bX, sem):
        wid = lax.axis_index("s") * 2 + lax.axis_index("c")
        base = wid * PER_W

        def body(ci, carry):
            off = base + ci * CH
            pltpu.sync_copy(dst_hbm.at[pl.ds(off, CH)], idxd)
            pltpu.sync_copy(src_hbm.at[pl.ds(off, CH)], idxs)
            c1 = pltpu.async_copy(Pp_hbm.at[idxd], bP, sem)
            c2 = pltpu.async_copy(Qp_hbm.at[idxs], bQ, sem)
            c3 = pltpu.async_copy(xv_hbm.at[idxs], bX, sem)
            c1.wait(); c2.wait(); c3.wait()
            pltpu.sync_copy(bP, Pd_hbm.at[pl.ds(off, CH)])
            pltpu.sync_copy(bQ, Qs_hbm.at[pl.ds(off, CH)])
            pltpu.sync_copy(bX, xvs_hbm.at[pl.ds(off, CH)])
            return carry

        lax.fori_loop(0, NCH, body, 0)

    return k(dstA, srcA, Pp, Qp, xv)


def _scatter_call(dstA, ex, evd, zeros_nd):
    mesh = plsc.VectorSubcoreMesh(core_axis_name="c", subcore_axis_name="s")

    @functools.partial(
        pl.kernel,
        mesh=mesh,
        out_type=[
            jax.ShapeDtypeStruct((N_PAD, D), _f32),
            jax.ShapeDtypeStruct((N_PAD, D), _f32),
        ],
        scratch_types=[
            pltpu.VMEM((CH,), jnp.int32),
            pltpu.VMEM((CH, D), _f32),
            pltpu.VMEM_SHARED((N_PAD, D), _f32),
        ],
    )
    def k(dst_hbm, ex_hbm, evd_hbm, z_hbm, s_hbm, num_hbm, idx, rows, acc):
        c = lax.axis_index("c")
        sid = lax.axis_index("s")
        pltpu.sync_copy(z_hbm.at[pl.ds(sid * NROW, NROW)],
                        acc.at[pl.ds(sid * NROW, NROW)])
        plsc.subcore_barrier()

        def run(data_hbm):
            def body(ci, carry):
                off = sid * PER_T + ci * CH
                pltpu.sync_copy(dst_hbm.at[pl.ds(off, CH)], idx)
                pltpu.sync_copy(data_hbm.at[pl.ds(off, CH)], rows)
                pltpu.sync_copy(rows, acc.at[idx], add=True)
                return carry
            lax.fori_loop(0, NCH_S, body, 0)

        @pl.when(c == 0)
        def _():
            run(ex_hbm)

        @pl.when(c == 1)
        def _():
            run(evd_hbm)

        plsc.subcore_barrier()

        @pl.when(c == 0)
        def _():
            pltpu.sync_copy(acc.at[pl.ds(sid * NROW, NROW)],
                            s_hbm.at[pl.ds(sid * NROW, NROW)])

        @pl.when(c == 1)
        def _():
            pltpu.sync_copy(acc.at[pl.ds(sid * NROW, NROW)],
                            num_hbm.at[pl.ds(sid * NROW, NROW)])

    return k(dstA, ex, evd, zeros_nd)


def _bmm(a, b):
    return jax.lax.dot(a.astype(jnp.bfloat16), b.astype(jnp.bfloat16),
                       preferred_element_type=jnp.float32)


def _espec(cols, colblk=0):
    return pl.BlockSpec((TE, cols), lambda i, _c=colblk: (i, _c))


def _acc_init(ref):
    @pl.when(pl.program_id(0) == 0)
    def _():
        ref[...] = jnp.zeros_like(ref)


def _t1_body(posd_ref, poss_ref, w_ref, pW1_ref, pb1_ref,
             s_ref, ss_ref, cnt_ref):
    rel = posd_ref[...] - poss_ref[...]
    h1 = rel @ pW1_ref[...] + pb1_ref[...]
    w = w_ref[...]
    wh = h1 * w
    _acc_init(s_ref); _acc_init(ss_ref); _acc_init(cnt_ref)
    s_ref[...] += jnp.sum(wh, axis=0)[None, :]
    ss_ref[...] += jnp.sum(wh * h1, axis=0)[None, :]
    cnt_ref[...] += jnp.full((1, D), jnp.sum(w), _f32)


def _t2_body(posd_ref, poss_ref, w_ref, pW1_ref, pb1_ref, sc1_ref, t1_ref,
             pW2_ref, pb2_ref, s_ref, ss_ref):
    rel = posd_ref[...] - poss_ref[...]
    h1 = rel @ pW1_ref[...] + pb1_ref[...]
    r = jnp.maximum(h1 * sc1_ref[...] + t1_ref[...], 0.0)
    h2 = _bmm(r, pW2_ref[...]) + pb2_ref[...]
    w = w_ref[...]
    wh = h2 * w
    _acc_init(s_ref); _acc_init(ss_ref)
    s_ref[...] += jnp.sum(wh, axis=0)[None, :]
    ss_ref[...] += jnp.sum(wh * h2, axis=0)[None, :]


def _delta(posd_ref, poss_ref, pW1_ref, pb1_ref, sc1_ref, t1_ref,
           pW2_ref, pb2_ref, sc2_ref, t2_ref):
    rel = posd_ref[...] - poss_ref[...]
    h1 = rel @ pW1_ref[...] + pb1_ref[...]
    r = jnp.maximum(h1 * sc1_ref[...] + t1_ref[...], 0.0)
    h2 = _bmm(r, pW2_ref[...]) + pb2_ref[...]
    return jnp.maximum(h2 * sc2_ref[...] + t2_ref[...], 0.0)


def _t3_body(posd_ref, poss_ref, Pd_ref, Qs_ref, w_ref,
             pW1_ref, pb1_ref, sc1_ref, t1_ref, pW2_ref, pb2_ref,
             sc2_ref, t2_ref, aW1_ref, ab1_ref,
             a1p_ref, s_ref, ss_ref):
    delta = _delta(posd_ref, poss_ref, pW1_ref, pb1_ref, sc1_ref, t1_ref,
                   pW2_ref, pb2_ref, sc2_ref, t2_ref)
    a1p = (Pd_ref[...] - Qs_ref[...]
           + _bmm(delta, aW1_ref[...]) + ab1_ref[...])
    a1p_ref[...] = a1p
    w = w_ref[...]
    wh = a1p * w
    _acc_init(s_ref); _acc_init(ss_ref)
    s_ref[...] += jnp.sum(wh, axis=0)[None, :]
    ss_ref[...] += jnp.sum(wh * a1p, axis=0)[None, :]


def _t4_body(a1p_ref, w_ref, sc3_ref, t3_ref, aW2_ref, ab2_ref,
             a2p_ref, s_ref, ss_ref, mx_ref, mn_ref):
    ra = jnp.maximum(a1p_ref[...] * sc3_ref[...] + t3_ref[...], 0.0)
    a2p = _bmm(ra, aW2_ref[...]) + ab2_ref[...]
    a2p_ref[...] = a2p
    w = w_ref[...]
    wh = a2p * w
    _acc_init(s_ref); _acc_init(ss_ref)
    s_ref[...] += jnp.sum(wh, axis=0)[None, :]
    ss_ref[...] += jnp.sum(wh * a2p, axis=0)[None, :]
    pmx = jnp.max(a2p, axis=0)[None, :]
    pmn = jnp.min(a2p, axis=0)[None, :]

    @pl.when(pl.program_id(0) == 0)
    def _():
        mx_ref[...] = pmx
        mn_ref[...] = pmn

    @pl.when(pl.program_id(0) > 0)
    def _():
        mx_ref[...] = jnp.maximum(mx_ref[...], pmx)
        mn_ref[...] = jnp.minimum(mn_ref[...], pmn)


def _t5_body(a2p_ref, posd_ref, poss_ref, xvs_ref, w_ref,
             pW1_ref, pb1_ref, sc1_ref, t1_ref, pW2_ref, pb2_ref,
             sc2_ref, t2_ref, sc4_ref, t4_ref, M_ref,
             ex_ref, evd_ref):
    delta = _delta(posd_ref, poss_ref, pW1_ref, pb1_ref, sc1_ref, t1_ref,
                   pW2_ref, pb2_ref, sc2_ref, t2_ref)
    af = jnp.maximum(a2p_ref[...] * sc4_ref[...] + t4_ref[...], 0.0)
    ex = w_ref[...] * jnp.exp(af - M_ref[...])
    ex_ref[...] = ex
    evd_ref[...] = ex * (xvs_ref[...] + delta)


def _t1_call(posd, poss, w2, pW1p, pb1):
    return pl.pallas_call(
        _t1_body,
        grid=(GRID_E,),
        in_specs=[_espec(16), _espec(16), _espec(1),
                  pl.BlockSpec((16, D), lambda i: (0, 0)), _rspec()],
        out_specs=[_rspec(), _rspec(), _rspec()],
        out_shape=[jax.ShapeDtypeStruct((1, D), _f32)] * 3,
    )(posd, poss, w2, pW1p, pb1)


def _t2_call(posd, poss, w2, pW1p, pb1, sc1, t1, pW2, pb2):
    return pl.pallas_call(
        _t2_body,
        grid=(GRID_E,),
        in_specs=[_espec(16), _espec(16), _espec(1),
                  pl.BlockSpec((16, D), lambda i: (0, 0)), _rspec(),
                  _rspec(), _rspec(), _wspec(), _rspec()],
        out_specs=[_rspec(), _rspec()],
        out_shape=[jax.ShapeDtypeStruct((1, D), _f32)] * 2,
    )(posd, poss, w2, pW1p, pb1, sc1, t1, pW2, pb2)


def _t3_call(posd, poss, Pd, Qs, w2, pW1p, pb1, sc1, t1, pW2, pb2,
             sc2, t2, aW1, ab1):
    return pl.pallas_call(
        _t3_body,
        grid=(GRID_E,),
        in_specs=[_espec(16), _espec(16), _espec(D), _espec(D), _espec(1),
                  pl.BlockSpec((16, D), lambda i: (0, 0)), _rspec(),
                  _rspec(), _rspec(), _wspec(), _rspec(),
                  _rspec(), _rspec(), _wspec(), _rspec()],
        out_specs=[_espec(D), _rspec(), _rspec()],
        out_shape=[jax.ShapeDtypeStruct((EP, D), _f32),
                   jax.ShapeDtypeStruct((1, D), _f32),
                   jax.ShapeDtypeStruct((1, D), _f32)],
    )(posd, poss, Pd, Qs, w2, pW1p, pb1, sc1, t1, pW2, pb2, sc2, t2, aW1, ab1)


def _t4_call(a1p, w2, sc3, t3, aW2, ab2):
    return pl.pallas_call(
        _t4_body,
        grid=(GRID_E,),
        in_specs=[_espec(D), _espec(1), _rspec(), _rspec(), _wspec(), _rspec()],
        out_specs=[_espec(D), _rspec(), _rspec(), _rspec(), _rspec()],
        out_shape=[jax.ShapeDtypeStruct((EP, D), _f32)] +
                  [jax.ShapeDtypeStruct((1, D), _f32)] * 4,
    )(a1p, w2, sc3, t3, aW2, ab2)


def _t5_call(a2p, posd, poss, xvs, w2, pW1p, pb1, sc1, t1, pW2, pb2,
             sc2, t2, sc4, t4, M):
    return pl.pallas_call(
        _t5_body,
        grid=(GRID_E,),
        in_specs=[_espec(D), _espec(16), _espec(16), _espec(D), _espec(1),
                  pl.BlockSpec((16, D), lambda i: (0, 0)), _rspec(),
                  _rspec(), _rspec(), _wspec(), _rspec(),
                  _rspec(), _rspec(), _rspec(), _rspec(),
                  pl.BlockSpec((1, 1), lambda i: (0, 0))],
        out_specs=[_espec(D), _espec(D)],
        out_shape=[jax.ShapeDtypeStruct((EP, D), _f32)] * 2,
    )(a2p, posd, poss, xvs, w2, pW1p, pb1, sc1, t1, pW2, pb2, sc2, t2,
      sc4, t4, M)


def _final_body(s_ref, num_ref, x1_ref, Wup_ref, bup_ref, g_ref, b_ref, o_ref):
    o = jnp.maximum((num_ref[...] / s_ref[...]) @ Wup_ref[...] + bup_ref[...],
                    0.0)
    h = o + x1_ref[...]
    mu = jnp.mean(h, axis=1, keepdims=True)
    var = jnp.mean((h - mu) * (h - mu), axis=1, keepdims=True)
    o_ref[...] = (h - mu) / jnp.sqrt(var + 1e-5) * g_ref[...] + b_ref[...]


def _final_call(s, num, x1, W_up, b_up, ln_g, ln_b):
    nspec = pl.BlockSpec((NTILE, D), lambda i: (i, 0))
    return pl.pallas_call(
        _final_body,
        grid=(GRID_N,),
        in_specs=[nspec, nspec, nspec, _wspec(), _rspec(), _rspec(), _rspec()],
        out_specs=nspec,
        out_shape=jax.ShapeDtypeStruct((N, D), _f32),
    )(s, num, x1, W_up, b_up, ln_g, ln_b)


def _bn_params(s, ss, cnt, g, b):
    mu = s / cnt
    var = ss / cnt - mu * mu
    sc = g[None, :] / jnp.sqrt(var + 1e-5)
    t = b[None, :] - mu * sc
    return sc, t


def kernel(x, pos, edge_index, W_in, b_in, W_lin, W_src, W_dst, pW1, pb1,
           pg1, pB1, pW2, pb2, pg2, pB2, aW1, ab1, ag1, aB1, aW2, ab2,
           ag2, aB2, W_up, b_up, ln_g, ln_b):
    src = edge_index[0]
    dst = edge_index[1]
    loop = jnp.arange(N, dtype=src.dtype)
    pad = EP - (E + N)
    srcA = jnp.concatenate([src, loop, jnp.zeros((pad,), src.dtype)])
    dstA = jnp.concatenate([dst, loop, jnp.zeros((pad,), src.dtype)])
    w2 = jnp.concatenate([(src != dst).astype(_f32), jnp.ones((N,), _f32),
                          jnp.zeros((pad,), _f32)]).reshape(EP, 1)
    pos16 = jnp.pad(pos, ((0, 0), (0, 13)))
    pW1p = jnp.pad(pW1, ((0, 13), (0, 0)))
    b_in2 = b_in.reshape(1, D)
    pb1r = pb1.reshape(1, D)
    pb2r = pb2.reshape(1, D)
    ab1r = ab1.reshape(1, D)
    ab2r = ab2.reshape(1, D)

    posd, poss = _gather_pos_call(dstA, srcA, pos16)

    x1, xv, Pp, Qp = _node_call(x, W_in, b_in2, W_lin, W_dst, W_src, aW1)

    Pd, Qs, xvs = _gather_main_call(dstA, srcA, Pp, Qp, xv)

    s1, ss1, cntv = _t1_call(posd, poss, w2, pW1p, pb1r)
    cnt = cntv[0, 0]
    sc1, t1 = _bn_params(s1, ss1, cnt, pg1, pB1)

    s2, ss2 = _t2_call(posd, poss, w2, pW1p, pb1r, sc1, t1, pW2, pb2r)
    sc2, t2 = _bn_params(s2, ss2, cnt, pg2, pB2)

    a1p, s3, ss3 = _t3_call(posd, poss, Pd, Qs, w2, pW1p, pb1r, sc1, t1,
                            pW2, pb2r, sc2, t2, aW1, ab1r)
    sc3, t3 = _bn_params(s3, ss3, cnt, ag1, aB1)

    a2p, s4, ss4, mxc, mnc = _t4_call(a1p, w2, sc3, t3, aW2, ab2r)
    sc4, t4 = _bn_params(s4, ss4, cnt, ag2, aB2)
    M = jnp.maximum(jnp.maximum(mxc * sc4 + t4, mnc * sc4 + t4), 0.0).max()

    ex, evd = _t5_call(a2p, posd, poss, xvs, w2, pW1p, pb1r, sc1, t1,
                       pW2, pb2r, sc2, t2, sc4, t4, M.reshape(1, 1))

    zeros_nd = jnp.zeros((N_PAD, D), _f32)
    ssum, num = _scatter_call(dstA, ex, evd, zeros_nd)
    ssum = ssum[:N]
    num = num[:N]

    return _final_call(ssum, num, x1, W_up, b_up.reshape(1, D), ln_g.reshape(1, D),
                       ln_b.reshape(1, D))


# speedup vs baseline: 1.3394x; 1.0106x over previous
import functools
import jax
import jax.numpy as jnp
from jax import lax
from jax.experimental import pallas as pl
from jax.experimental.pallas import tpu as pltpu
from jax.experimental.pallas import tpu_sc as plsc

N = 10000
E = 320000
D = 128
EP = 331776
TE = 4096
GRID_E = EP // TE
NTILE = 1000
GRID_N = N // NTILE

NW = 32
PER_W = EP // NW
CH = 128
NCH = PER_W // CH
PER_T = EP // 16
NCH_S = PER_T // CH
N_PAD = 10240
NROW = N_PAD // 16

_f32 = jnp.float32


def _node_body(x_ref, Win_ref, bin_ref, Wlin_ref, Wd_ref, Ws_ref, aW1_ref,
               x1_ref, xv_ref, Pp_ref, Qp_ref):
    x1 = jnp.maximum(x_ref[...] @ Win_ref[...] + bin_ref[...], 0.0)
    x1_ref[...] = x1
    xv_ref[...] = x1 @ Wlin_ref[...]
    Pp_ref[...] = x1 @ (Wd_ref[...] @ aW1_ref[...])
    Qp_ref[...] = x1 @ (Ws_ref[...] @ aW1_ref[...])


def _wspec():
    return pl.BlockSpec((D, D), lambda i: (0, 0))


def _rspec():
    return pl.BlockSpec((1, D), lambda i: (0, 0))


def _node_call(x, W_in, b_in, W_lin, W_d, W_s, aW1):
    nspec = pl.BlockSpec((NTILE, D), lambda i: (i, 0))
    outs = [jax.ShapeDtypeStruct((N, D), _f32)] * 4
    return pl.pallas_call(
        _node_body,
        grid=(GRID_N,),
        in_specs=[nspec, _wspec(), _rspec(), _wspec(), _wspec(), _wspec(), _wspec()],
        out_specs=[nspec] * 4,
        out_shape=outs,
    )(x, W_in, b_in, W_lin, W_d, W_s, aW1)


def _gather_pos_call(dstA, srcA, pos16):
    mesh = plsc.VectorSubcoreMesh(core_axis_name="c", subcore_axis_name="s")

    @functools.partial(
        pl.kernel,
        mesh=mesh,
        out_type=[
            jax.ShapeDtypeStruct((EP, 16), _f32),
            jax.ShapeDtypeStruct((EP, 16), _f32),
        ],
        scratch_types=[
            pltpu.VMEM((CH,), jnp.int32),
            pltpu.VMEM((CH,), jnp.int32),
            pltpu.VMEM((CH, 16), _f32),
            pltpu.VMEM((CH, 16), _f32),
            pltpu.SemaphoreType.DMA,
        ],
        compiler_params=pltpu.CompilerParams(use_tc_tiling_on_sc=False),
    )
    def k(dst_hbm, src_hbm, pos_hbm, posd_hbm, poss_hbm,
          idxd, idxs, bpd, bps, sem):
        wid = lax.axis_index("s") * 2 + lax.axis_index("c")
        base = wid * PER_W

        def body(ci, carry):
            off = base + ci * CH
            pltpu.sync_copy(dst_hbm.at[pl.ds(off, CH)], idxd)
            pltpu.sync_copy(src_hbm.at[pl.ds(off, CH)], idxs)
            c4 = pltpu.async_copy(pos_hbm.at[idxd], bpd, sem)
            c5 = pltpu.async_copy(pos_hbm.at[idxs], bps, sem)
            c4.wait(); c5.wait()
            pltpu.sync_copy(bpd, posd_hbm.at[pl.ds(off, CH)])
            pltpu.sync_copy(bps, poss_hbm.at[pl.ds(off, CH)])
            return carry

        lax.fori_loop(0, NCH, body, 0)

    return k(dstA, srcA, pos16)


def _gather_main_call(dstA, srcA, Pp, Qp, xv):
    mesh = plsc.VectorSubcoreMesh(core_axis_name="c", subcore_axis_name="s")

    @functools.partial(
        pl.kernel,
        mesh=mesh,
        out_type=[
            jax.ShapeDtypeStruct((EP, D), _f32),
            jax.ShapeDtypeStruct((EP, D), _f32),
            jax.ShapeDtypeStruct((EP, D), _f32),
        ],
        scratch_types=[
            pltpu.VMEM((CH,), jnp.int32),
            pltpu.VMEM((CH,), jnp.int32),
            pltpu.VMEM((CH, D), _f32),
            pltpu.VMEM((CH, D), _f32),
            pltpu.VMEM((CH, D), _f32),
            pltpu.SemaphoreType.DMA,
        ],
        compiler_params=pltpu.CompilerParams(use_tc_tiling_on_sc=False),
    )
    def k(dst_hbm, src_hbm, Pp_hbm, Qp_hbm, xv_hbm,
          Pd_hbm, Qs_hbm, xvs_hbm,
          idxd, idxs, bP, bQ, bX, sem):
        wid = lax.axis_index("s") * 2 + lax.axis_index("c")
        base = wid * PER_W

        def body(ci, carry):
            off = base + ci * CH
            pltpu.sync_copy(dst_hbm.at[pl.ds(off, CH)], idxd)
            pltpu.sync_copy(src_hbm.at[pl.ds(off, CH)], idxs)
            c1 = pltpu.async_copy(Pp_hbm.at[idxd], bP, sem)
            c2 = pltpu.async_copy(Qp_hbm.at[idxs], bQ, sem)
            c3 = pltpu.async_copy(xv_hbm.at[idxs], bX, sem)
            c1.wait(); c2.wait(); c3.wait()
            pltpu.sync_copy(bP, Pd_hbm.at[pl.ds(off, CH)])
            pltpu.sync_copy(bQ, Qs_hbm.at[pl.ds(off, CH)])
            pltpu.sync_copy(bX, xvs_hbm.at[pl.ds(off, CH)])
            return carry

        lax.fori_loop(0, NCH, body, 0)

    return k(dstA, srcA, Pp, Qp, xv)


def _scatter_call(dstA, ex, evd, zeros_nd):
    mesh = plsc.VectorSubcoreMesh(core_axis_name="c", subcore_axis_name="s")

    @functools.partial(
        pl.kernel,
        mesh=mesh,
        out_type=[
            jax.ShapeDtypeStruct((N_PAD, D), _f32),
            jax.ShapeDtypeStruct((N_PAD, D), _f32),
        ],
        scratch_types=[
            pltpu.VMEM((CH,), jnp.int32),
            pltpu.VMEM((CH, D), _f32),
            pltpu.VMEM_SHARED((N_PAD, D), _f32),
        ],
    )
    def k(dst_hbm, ex_hbm, evd_hbm, z_hbm, s_hbm, num_hbm, idx, rows, acc):
        c = lax.axis_index("c")
        sid = lax.axis_index("s")
        pltpu.sync_copy(z_hbm.at[pl.ds(sid * NROW, NROW)],
                        acc.at[pl.ds(sid * NROW, NROW)])
        plsc.subcore_barrier()

        def run(data_hbm):
            def body(ci, carry):
                off = sid * PER_T + ci * CH
                pltpu.sync_copy(dst_hbm.at[pl.ds(off, CH)], idx)
                pltpu.sync_copy(data_hbm.at[pl.ds(off, CH)], rows)
                pltpu.sync_copy(rows, acc.at[idx], add=True)
                return carry
            lax.fori_loop(0, NCH_S, body, 0)

        @pl.when(c == 0)
        def _():
            run(ex_hbm)

        @pl.when(c == 1)
        def _():
            run(evd_hbm)

        plsc.subcore_barrier()

        @pl.when(c == 0)
        def _():
            pltpu.sync_copy(acc.at[pl.ds(sid * NROW, NROW)],
                            s_hbm.at[pl.ds(sid * NROW, NROW)])

        @pl.when(c == 1)
        def _():
            pltpu.sync_copy(acc.at[pl.ds(sid * NROW, NROW)],
                            num_hbm.at[pl.ds(sid * NROW, NROW)])

    return k(dstA, ex, evd, zeros_nd)


def _bmm(a, b):
    return jax.lax.dot(a.astype(jnp.bfloat16), b.astype(jnp.bfloat16),
                       preferred_element_type=jnp.float32)


def _espec(cols, colblk=0):
    return pl.BlockSpec((TE, cols), lambda i, _c=colblk: (i, _c))


def _acc_init(ref):
    @pl.when(pl.program_id(0) == 0)
    def _():
        ref[...] = jnp.zeros_like(ref)


def _t1_body(posd_ref, poss_ref, w_ref, pW1_ref, pb1_ref,
             s_ref, ss_ref, cnt_ref):
    rel = posd_ref[...] - poss_ref[...]
    h1 = rel @ pW1_ref[...] + pb1_ref[...]
    w = w_ref[...]
    wh = h1 * w
    _acc_init(s_ref); _acc_init(ss_ref); _acc_init(cnt_ref)
    s_ref[...] += jnp.sum(wh, axis=0)[None, :]
    ss_ref[...] += jnp.sum(wh * h1, axis=0)[None, :]
    cnt_ref[...] += jnp.full((1, D), jnp.sum(w), _f32)


def _t2_body(posd_ref, poss_ref, w_ref, pW1_ref, pb1_ref, sc1_ref, t1_ref,
             pW2_ref, pb2_ref, s_ref, ss_ref):
    rel = posd_ref[...] - poss_ref[...]
    h1 = rel @ pW1_ref[...] + pb1_ref[...]
    r = jnp.maximum(h1 * sc1_ref[...] + t1_ref[...], 0.0)
    h2 = _bmm(r, pW2_ref[...]) + pb2_ref[...]
    w = w_ref[...]
    wh = h2 * w
    _acc_init(s_ref); _acc_init(ss_ref)
    s_ref[...] += jnp.sum(wh, axis=0)[None, :]
    ss_ref[...] += jnp.sum(wh * h2, axis=0)[None, :]


def _delta(posd_ref, poss_ref, pW1_ref, pb1_ref, sc1_ref, t1_ref,
           pW2_ref, pb2_ref, sc2_ref, t2_ref):
    rel = posd_ref[...] - poss_ref[...]
    h1 = rel @ pW1_ref[...] + pb1_ref[...]
    r = jnp.maximum(h1 * sc1_ref[...] + t1_ref[...], 0.0)
    h2 = _bmm(r, pW2_ref[...]) + pb2_ref[...]
    return jnp.maximum(h2 * sc2_ref[...] + t2_ref[...], 0.0)


def _t3_body(posd_ref, poss_ref, Pd_ref, Qs_ref, w_ref,
             pW1_ref, pb1_ref, sc1_ref, t1_ref, pW2_ref, pb2_ref,
             sc2_ref, t2_ref, aW1_ref, ab1_ref,
             a1p_ref, s_ref, ss_ref):
    delta = _delta(posd_ref, poss_ref, pW1_ref, pb1_ref, sc1_ref, t1_ref,
                   pW2_ref, pb2_ref, sc2_ref, t2_ref)
    a1p = (Pd_ref[...] - Qs_ref[...]
           + _bmm(delta, aW1_ref[...]) + ab1_ref[...])
    a1p_ref[...] = a1p
    w = w_ref[...]
    wh = a1p * w
    _acc_init(s_ref); _acc_init(ss_ref)
    s_ref[...] += jnp.sum(wh, axis=0)[None, :]
    ss_ref[...] += jnp.sum(wh * a1p, axis=0)[None, :]


def _t4_body(a1p_ref, w_ref, sc3_ref, t3_ref, aW2_ref, ab2_ref,
             s_ref, ss_ref, mx_ref, mn_ref):
    ra = jnp.maximum(a1p_ref[...] * sc3_ref[...] + t3_ref[...], 0.0)
    a2p = _bmm(ra, aW2_ref[...]) + ab2_ref[...]
    w = w_ref[...]
    wh = a2p * w
    _acc_init(s_ref); _acc_init(ss_ref)
    s_ref[...] += jnp.sum(wh, axis=0)[None, :]
    ss_ref[...] += jnp.sum(wh * a2p, axis=0)[None, :]
    pmx = jnp.max(a2p, axis=0)[None, :]
    pmn = jnp.min(a2p, axis=0)[None, :]

    @pl.when(pl.program_id(0) == 0)
    def _():
        mx_ref[...] = pmx
        mn_ref[...] = pmn

    @pl.when(pl.program_id(0) > 0)
    def _():
        mx_ref[...] = jnp.maximum(mx_ref[...], pmx)
        mn_ref[...] = jnp.minimum(mn_ref[...], pmn)


def _t5_body(a1p_ref, posd_ref, poss_ref, xvs_ref, w_ref,
             pW1_ref, pb1_ref, sc1_ref, t1_ref, pW2_ref, pb2_ref,
             sc2_ref, t2_ref, sc3_ref, t3_ref, aW2_ref, ab2_ref,
             sc4_ref, t4_ref, M_ref,
             ex_ref, evd_ref):
    delta = _delta(posd_ref, poss_ref, pW1_ref, pb1_ref, sc1_ref, t1_ref,
                   pW2_ref, pb2_ref, sc2_ref, t2_ref)
    ra = jnp.maximum(a1p_ref[...] * sc3_ref[...] + t3_ref[...], 0.0)
    a2p = _bmm(ra, aW2_ref[...]) + ab2_ref[...]
    af = jnp.maximum(a2p * sc4_ref[...] + t4_ref[...], 0.0)
    ex = w_ref[...] * jnp.exp(af - M_ref[...])
    ex_ref[...] = ex
    evd_ref[...] = ex * (xvs_ref[...] + delta)


def _t1_call(posd, poss, w2, pW1p, pb1):
    return pl.pallas_call(
        _t1_body,
        grid=(GRID_E,),
        in_specs=[_espec(16), _espec(16), _espec(1),
                  pl.BlockSpec((16, D), lambda i: (0, 0)), _rspec()],
        out_specs=[_rspec(), _rspec(), _rspec()],
        out_shape=[jax.ShapeDtypeStruct((1, D), _f32)] * 3,
    )(posd, poss, w2, pW1p, pb1)


def _t2_call(posd, poss, w2, pW1p, pb1, sc1, t1, pW2, pb2):
    return pl.pallas_call(
        _t2_body,
        grid=(GRID_E,),
        in_specs=[_espec(16), _espec(16), _espec(1),
                  pl.BlockSpec((16, D), lambda i: (0, 0)), _rspec(),
                  _rspec(), _rspec(), _wspec(), _rspec()],
        out_specs=[_rspec(), _rspec()],
        out_shape=[jax.ShapeDtypeStruct((1, D), _f32)] * 2,
    )(posd, poss, w2, pW1p, pb1, sc1, t1, pW2, pb2)


def _t3_call(posd, poss, Pd, Qs, w2, pW1p, pb1, sc1, t1, pW2, pb2,
             sc2, t2, aW1, ab1):
    return pl.pallas_call(
        _t3_body,
        grid=(GRID_E,),
        in_specs=[_espec(16), _espec(16), _espec(D), _espec(D), _espec(1),
                  pl.BlockSpec((16, D), lambda i: (0, 0)), _rspec(),
                  _rspec(), _rspec(), _wspec(), _rspec(),
                  _rspec(), _rspec(), _wspec(), _rspec()],
        out_specs=[_espec(D), _rspec(), _rspec()],
        out_shape=[jax.ShapeDtypeStruct((EP, D), _f32),
                   jax.ShapeDtypeStruct((1, D), _f32),
                   jax.ShapeDtypeStruct((1, D), _f32)],
    )(posd, poss, Pd, Qs, w2, pW1p, pb1, sc1, t1, pW2, pb2, sc2, t2, aW1, ab1)


def _t4_call(a1p, w2, sc3, t3, aW2, ab2):
    return pl.pallas_call(
        _t4_body,
        grid=(GRID_E,),
        in_specs=[_espec(D), _espec(1), _rspec(), _rspec(), _wspec(), _rspec()],
        out_specs=[_rspec(), _rspec(), _rspec(), _rspec()],
        out_shape=[jax.ShapeDtypeStruct((1, D), _f32)] * 4,
    )(a1p, w2, sc3, t3, aW2, ab2)


def _t5_call(a1p, posd, poss, xvs, w2, pW1p, pb1, sc1, t1, pW2, pb2,
             sc2, t2, sc3, t3, aW2, ab2, sc4, t4, M):
    return pl.pallas_call(
        _t5_body,
        grid=(GRID_E,),
        in_specs=[_espec(D), _espec(16), _espec(16), _espec(D), _espec(1),
                  pl.BlockSpec((16, D), lambda i: (0, 0)), _rspec(),
                  _rspec(), _rspec(), _wspec(), _rspec(),
                  _rspec(), _rspec(), _rspec(), _rspec(),
                  _wspec(), _rspec(), _rspec(), _rspec(),
                  pl.BlockSpec((1, 1), lambda i: (0, 0))],
        out_specs=[_espec(D), _espec(D)],
        out_shape=[jax.ShapeDtypeStruct((EP, D), _f32)] * 2,
    )(a1p, posd, poss, xvs, w2, pW1p, pb1, sc1, t1, pW2, pb2, sc2, t2,
      sc3, t3, aW2, ab2, sc4, t4, M)


def _final_body(s_ref, num_ref, x1_ref, Wup_ref, bup_ref, g_ref, b_ref, o_ref):
    o = jnp.maximum((num_ref[...] / s_ref[...]) @ Wup_ref[...] + bup_ref[...],
                    0.0)
    h = o + x1_ref[...]
    mu = jnp.mean(h, axis=1, keepdims=True)
    var = jnp.mean((h - mu) * (h - mu), axis=1, keepdims=True)
    o_ref[...] = (h - mu) / jnp.sqrt(var + 1e-5) * g_ref[...] + b_ref[...]


def _final_call(s, num, x1, W_up, b_up, ln_g, ln_b):
    nspec = pl.BlockSpec((NTILE, D), lambda i: (i, 0))
    return pl.pallas_call(
        _final_body,
        grid=(GRID_N,),
        in_specs=[nspec, nspec, nspec, _wspec(), _rspec(), _rspec(), _rspec()],
        out_specs=nspec,
        out_shape=jax.ShapeDtypeStruct((N, D), _f32),
    )(s, num, x1, W_up, b_up, ln_g, ln_b)


def _bn_params(s, ss, cnt, g, b):
    mu = s / cnt
    var = ss / cnt - mu * mu
    sc = g[None, :] / jnp.sqrt(var + 1e-5)
    t = b[None, :] - mu * sc
    return sc, t


def kernel(x, pos, edge_index, W_in, b_in, W_lin, W_src, W_dst, pW1, pb1,
           pg1, pB1, pW2, pb2, pg2, pB2, aW1, ab1, ag1, aB1, aW2, ab2,
           ag2, aB2, W_up, b_up, ln_g, ln_b):
    src = edge_index[0]
    dst = edge_index[1]
    loop = jnp.arange(N, dtype=src.dtype)
    pad = EP - (E + N)
    srcA = jnp.concatenate([src, loop, jnp.zeros((pad,), src.dtype)])
    dstA = jnp.concatenate([dst, loop, jnp.zeros((pad,), src.dtype)])
    w2 = jnp.concatenate([(src != dst).astype(_f32), jnp.ones((N,), _f32),
                          jnp.zeros((pad,), _f32)]).reshape(EP, 1)
    pos16 = jnp.pad(pos, ((0, 0), (0, 13)))
    pW1p = jnp.pad(pW1, ((0, 13), (0, 0)))
    b_in2 = b_in.reshape(1, D)
    pb1r = pb1.reshape(1, D)
    pb2r = pb2.reshape(1, D)
    ab1r = ab1.reshape(1, D)
    ab2r = ab2.reshape(1, D)

    posd, poss = _gather_pos_call(dstA, srcA, pos16)

    x1, xv, Pp, Qp = _node_call(x, W_in, b_in2, W_lin, W_dst, W_src, aW1)

    Pd, Qs, xvs = _gather_main_call(dstA, srcA, Pp, Qp, xv)

    s1, ss1, cntv = _t1_call(posd, poss, w2, pW1p, pb1r)
    cnt = cntv[0, 0]
    sc1, t1 = _bn_params(s1, ss1, cnt, pg1, pB1)

    s2, ss2 = _t2_call(posd, poss, w2, pW1p, pb1r, sc1, t1, pW2, pb2r)
    sc2, t2 = _bn_params(s2, ss2, cnt, pg2, pB2)

    a1p, s3, ss3 = _t3_call(posd, poss, Pd, Qs, w2, pW1p, pb1r, sc1, t1,
                            pW2, pb2r, sc2, t2, aW1, ab1r)
    sc3, t3 = _bn_params(s3, ss3, cnt, ag1, aB1)

    s4, ss4, mxc, mnc = _t4_call(a1p, w2, sc3, t3, aW2, ab2r)
    sc4, t4 = _bn_params(s4, ss4, cnt, ag2, aB2)
    M = jnp.maximum(jnp.maximum(mxc * sc4 + t4, mnc * sc4 + t4), 0.0).max()

    ex, evd = _t5_call(a1p, posd, poss, xvs, w2, pW1p, pb1r, sc1, t1,
                       pW2, pb2r, sc2, t2, sc3, t3, aW2, ab2r,
                       sc4, t4, M.reshape(1, 1))

    zeros_nd = jnp.zeros((N_PAD, D), _f32)
    ssum, num = _scatter_call(dstA, ex, evd, zeros_nd)
    ssum = ssum[:N]
    num = num[:N]

    return _final_call(ssum, num, x1, W_up, b_up.reshape(1, D), ln_g.reshape(1, D),
                       ln_b.reshape(1, D))


# speedup vs baseline: 1.3862x; 1.0349x over previous
import functools
import jax
import jax.numpy as jnp
from jax import lax
from jax.experimental import pallas as pl
from jax.experimental.pallas import tpu as pltpu
from jax.experimental.pallas import tpu_sc as plsc

N = 10000
E = 320000
D = 128
EP = 331776
TE = 4096
GRID_E = EP // TE
NTILE = 1000
GRID_N = N // NTILE

NW = 32
PER_W = EP // NW
CH = 128
NCH = PER_W // CH
GRID_A = 41
GRID_B = GRID_E - GRID_A
EP_A = GRID_A * TE
EP_B = EP - EP_A
N_PAD = 10240
NROW = N_PAD // 16

_f32 = jnp.float32


def _node_body(x_ref, Win_ref, bin_ref, Wlin_ref, Wd_ref, Ws_ref, aW1_ref,
               x1_ref, xv_ref, Pp_ref, Qp_ref):
    x1 = jnp.maximum(x_ref[...] @ Win_ref[...] + bin_ref[...], 0.0)
    x1_ref[...] = x1
    xv_ref[...] = x1 @ Wlin_ref[...]
    Pp_ref[...] = x1 @ (Wd_ref[...] @ aW1_ref[...])
    Qp_ref[...] = x1 @ (Ws_ref[...] @ aW1_ref[...])


def _wspec():
    return pl.BlockSpec((D, D), lambda i: (0, 0))


def _rspec():
    return pl.BlockSpec((1, D), lambda i: (0, 0))


def _node_call(x, W_in, b_in, W_lin, W_d, W_s, aW1):
    nspec = pl.BlockSpec((NTILE, D), lambda i: (i, 0))
    outs = [jax.ShapeDtypeStruct((N, D), _f32)] * 4
    return pl.pallas_call(
        _node_body,
        grid=(GRID_N,),
        in_specs=[nspec, _wspec(), _rspec(), _wspec(), _wspec(), _wspec(), _wspec()],
        out_specs=[nspec] * 4,
        out_shape=outs,
    )(x, W_in, b_in, W_lin, W_d, W_s, aW1)


def _gather_pos_call(dstA, srcA, pos16):
    mesh = plsc.VectorSubcoreMesh(core_axis_name="c", subcore_axis_name="s")

    @functools.partial(
        pl.kernel,
        mesh=mesh,
        out_type=[
            jax.ShapeDtypeStruct((EP, 16), _f32),
            jax.ShapeDtypeStruct((EP, 16), _f32),
        ],
        scratch_types=[
            pltpu.VMEM((CH,), jnp.int32),
            pltpu.VMEM((CH,), jnp.int32),
            pltpu.VMEM((CH, 16), _f32),
            pltpu.VMEM((CH, 16), _f32),
            pltpu.SemaphoreType.DMA,
        ],
        compiler_params=pltpu.CompilerParams(use_tc_tiling_on_sc=False),
    )
    def k(dst_hbm, src_hbm, pos_hbm, posd_hbm, poss_hbm,
          idxd, idxs, bpd, bps, sem):
        wid = lax.axis_index("s") * 2 + lax.axis_index("c")
        base = wid * PER_W

        def body(ci, carry):
            off = base + ci * CH
            pltpu.sync_copy(dst_hbm.at[pl.ds(off, CH)], idxd)
            pltpu.sync_copy(src_hbm.at[pl.ds(off, CH)], idxs)
            c4 = pltpu.async_copy(pos_hbm.at[idxd], bpd, sem)
            c5 = pltpu.async_copy(pos_hbm.at[idxs], bps, sem)
            c4.wait(); c5.wait()
            pltpu.sync_copy(bpd, posd_hbm.at[pl.ds(off, CH)])
            pltpu.sync_copy(bps, poss_hbm.at[pl.ds(off, CH)])
            return carry

        lax.fori_loop(0, NCH, body, 0)

    return k(dstA, srcA, pos16)


def _gather_main_call(dstA, srcA, Pp, Qp, xv):
    mesh = plsc.VectorSubcoreMesh(core_axis_name="c", subcore_axis_name="s")

    @functools.partial(
        pl.kernel,
        mesh=mesh,
        out_type=[
            jax.ShapeDtypeStruct((EP, D), _f32),
            jax.ShapeDtypeStruct((EP, D), _f32),
            jax.ShapeDtypeStruct((EP, D), _f32),
        ],
        scratch_types=[
            pltpu.VMEM((CH,), jnp.int32),
            pltpu.VMEM((CH,), jnp.int32),
            pltpu.VMEM((CH, D), _f32),
            pltpu.VMEM((CH, D), _f32),
            pltpu.VMEM((CH, D), _f32),
            pltpu.SemaphoreType.DMA,
        ],
        compiler_params=pltpu.CompilerParams(use_tc_tiling_on_sc=False),
    )
    def k(dst_hbm, src_hbm, Pp_hbm, Qp_hbm, xv_hbm,
          Pd_hbm, Qs_hbm, xvs_hbm,
          idxd, idxs, bP, bQ, bX, sem):
        wid = lax.axis_index("s") * 2 + lax.axis_index("c")
        base = wid * PER_W

        def body(ci, carry):
            off = base + ci * CH
            pltpu.sync_copy(dst_hbm.at[pl.ds(off, CH)], idxd)
            pltpu.sync_copy(src_hbm.at[pl.ds(off, CH)], idxs)
            c1 = pltpu.async_copy(Pp_hbm.at[idxd], bP, sem)
            c2 = pltpu.async_copy(Qp_hbm.at[idxs], bQ, sem)
            c3 = pltpu.async_copy(xv_hbm.at[idxs], bX, sem)
            c1.wait(); c2.wait(); c3.wait()
            pltpu.sync_copy(bP, Pd_hbm.at[pl.ds(off, CH)])
            pltpu.sync_copy(bQ, Qs_hbm.at[pl.ds(off, CH)])
            pltpu.sync_copy(bX, xvs_hbm.at[pl.ds(off, CH)])
            return carry

        lax.fori_loop(0, NCH, body, 0)

    return k(dstA, srcA, Pp, Qp, xv)


def _scatter_call(dstA, ex, evd, zeros_nd):
    ep = ex.shape[0]
    per_t = ep // 16
    nch = per_t // CH
    mesh = plsc.VectorSubcoreMesh(core_axis_name="c", subcore_axis_name="s")

    @functools.partial(
        pl.kernel,
        mesh=mesh,
        out_type=[
            jax.ShapeDtypeStruct((N_PAD, D), _f32),
            jax.ShapeDtypeStruct((N_PAD, D), _f32),
        ],
        scratch_types=[
            pltpu.VMEM((CH,), jnp.int32),
            pltpu.VMEM((CH, D), _f32),
            pltpu.VMEM_SHARED((N_PAD, D), _f32),
        ],
    )
    def k(dst_hbm, ex_hbm, evd_hbm, z_hbm, s_hbm, num_hbm, idx, rows, acc):
        c = lax.axis_index("c")
        sid = lax.axis_index("s")
        pltpu.sync_copy(z_hbm.at[pl.ds(sid * NROW, NROW)],
                        acc.at[pl.ds(sid * NROW, NROW)])
        plsc.subcore_barrier()

        def run(data_hbm):
            def body(ci, carry):
                off = sid * per_t + ci * CH
                pltpu.sync_copy(dst_hbm.at[pl.ds(off, CH)], idx)
                pltpu.sync_copy(data_hbm.at[pl.ds(off, CH)], rows)
                pltpu.sync_copy(rows, acc.at[idx], add=True)
                return carry
            lax.fori_loop(0, nch, body, 0)

        @pl.when(c == 0)
        def _():
            run(ex_hbm)

        @pl.when(c == 1)
        def _():
            run(evd_hbm)

        plsc.subcore_barrier()

        @pl.when(c == 0)
        def _():
            pltpu.sync_copy(acc.at[pl.ds(sid * NROW, NROW)],
                            s_hbm.at[pl.ds(sid * NROW, NROW)])

        @pl.when(c == 1)
        def _():
            pltpu.sync_copy(acc.at[pl.ds(sid * NROW, NROW)],
                            num_hbm.at[pl.ds(sid * NROW, NROW)])

    return k(dstA, ex, evd, zeros_nd)


def _bmm(a, b):
    return jax.lax.dot(a.astype(jnp.bfloat16), b.astype(jnp.bfloat16),
                       preferred_element_type=jnp.float32)


def _espec(cols, colblk=0):
    return pl.BlockSpec((TE, cols), lambda i, _c=colblk: (i, _c))


def _acc_init(ref):
    @pl.when(pl.program_id(0) == 0)
    def _():
        ref[...] = jnp.zeros_like(ref)


def _t1_body(posd_ref, poss_ref, w_ref, pW1_ref, pb1_ref,
             s_ref, ss_ref, cnt_ref):
    rel = posd_ref[...] - poss_ref[...]
    h1 = rel @ pW1_ref[...] + pb1_ref[...]
    w = w_ref[...]
    wh = h1 * w
    _acc_init(s_ref); _acc_init(ss_ref); _acc_init(cnt_ref)
    s_ref[...] += jnp.sum(wh, axis=0)[None, :]
    ss_ref[...] += jnp.sum(wh * h1, axis=0)[None, :]
    cnt_ref[...] += jnp.full((1, D), jnp.sum(w), _f32)


def _t2_body(posd_ref, poss_ref, w_ref, pW1_ref, pb1_ref, sc1_ref, t1_ref,
             pW2_ref, pb2_ref, s_ref, ss_ref):
    rel = posd_ref[...] - poss_ref[...]
    h1 = rel @ pW1_ref[...] + pb1_ref[...]
    r = jnp.maximum(h1 * sc1_ref[...] + t1_ref[...], 0.0)
    h2 = _bmm(r, pW2_ref[...]) + pb2_ref[...]
    w = w_ref[...]
    wh = h2 * w
    _acc_init(s_ref); _acc_init(ss_ref)
    s_ref[...] += jnp.sum(wh, axis=0)[None, :]
    ss_ref[...] += jnp.sum(wh * h2, axis=0)[None, :]


def _delta(posd_ref, poss_ref, pW1_ref, pb1_ref, sc1_ref, t1_ref,
           pW2_ref, pb2_ref, sc2_ref, t2_ref):
    rel = posd_ref[...] - poss_ref[...]
    h1 = rel @ pW1_ref[...] + pb1_ref[...]
    r = jnp.maximum(h1 * sc1_ref[...] + t1_ref[...], 0.0)
    h2 = _bmm(r, pW2_ref[...]) + pb2_ref[...]
    return jnp.maximum(h2 * sc2_ref[...] + t2_ref[...], 0.0)


def _t3_body(posd_ref, poss_ref, Pd_ref, Qs_ref, w_ref,
             pW1_ref, pb1_ref, sc1_ref, t1_ref, pW2_ref, pb2_ref,
             sc2_ref, t2_ref, aW1_ref, ab1_ref,
             a1p_ref, s_ref, ss_ref):
    delta = _delta(posd_ref, poss_ref, pW1_ref, pb1_ref, sc1_ref, t1_ref,
                   pW2_ref, pb2_ref, sc2_ref, t2_ref)
    a1p = (Pd_ref[...] - Qs_ref[...]
           + _bmm(delta, aW1_ref[...]) + ab1_ref[...])
    a1p_ref[...] = a1p
    w = w_ref[...]
    wh = a1p * w
    _acc_init(s_ref); _acc_init(ss_ref)
    s_ref[...] += jnp.sum(wh, axis=0)[None, :]
    ss_ref[...] += jnp.sum(wh * a1p, axis=0)[None, :]


def _t4_body(a1p_ref, w_ref, sc3_ref, t3_ref, aW2_ref, ab2_ref,
             s_ref, ss_ref, mx_ref, mn_ref):
    ra = jnp.maximum(a1p_ref[...] * sc3_ref[...] + t3_ref[...], 0.0)
    a2p = _bmm(ra, aW2_ref[...]) + ab2_ref[...]
    w = w_ref[...]
    wh = a2p * w
    _acc_init(s_ref); _acc_init(ss_ref)
    s_ref[...] += jnp.sum(wh, axis=0)[None, :]
    ss_ref[...] += jnp.sum(wh * a2p, axis=0)[None, :]
    pmx = jnp.max(a2p, axis=0)[None, :]
    pmn = jnp.min(a2p, axis=0)[None, :]

    @pl.when(pl.program_id(0) == 0)
    def _():
        mx_ref[...] = pmx
        mn_ref[...] = pmn

    @pl.when(pl.program_id(0) > 0)
    def _():
        mx_ref[...] = jnp.maximum(mx_ref[...], pmx)
        mn_ref[...] = jnp.minimum(mn_ref[...], pmn)


def _t5_body(a1p_ref, posd_ref, poss_ref, xvs_ref, w_ref,
             pW1_ref, pb1_ref, sc1_ref, t1_ref, pW2_ref, pb2_ref,
             sc2_ref, t2_ref, sc3_ref, t3_ref, aW2_ref, ab2_ref,
             sc4_ref, t4_ref, M_ref,
             ex_ref, evd_ref):
    delta = _delta(posd_ref, poss_ref, pW1_ref, pb1_ref, sc1_ref, t1_ref,
                   pW2_ref, pb2_ref, sc2_ref, t2_ref)
    ra = jnp.maximum(a1p_ref[...] * sc3_ref[...] + t3_ref[...], 0.0)
    a2p = _bmm(ra, aW2_ref[...]) + ab2_ref[...]
    af = jnp.maximum(a2p * sc4_ref[...] + t4_ref[...], 0.0)
    ex = w_ref[...] * jnp.exp(af - M_ref[...])
    ex_ref[...] = ex
    evd_ref[...] = ex * (xvs_ref[...] + delta)


def _t1_call(posd, poss, w2, pW1p, pb1):
    return pl.pallas_call(
        _t1_body,
        grid=(GRID_E,),
        in_specs=[_espec(16), _espec(16), _espec(1),
                  pl.BlockSpec((16, D), lambda i: (0, 0)), _rspec()],
        out_specs=[_rspec(), _rspec(), _rspec()],
        out_shape=[jax.ShapeDtypeStruct((1, D), _f32)] * 3,
    )(posd, poss, w2, pW1p, pb1)


def _t2_call(posd, poss, w2, pW1p, pb1, sc1, t1, pW2, pb2):
    return pl.pallas_call(
        _t2_body,
        grid=(GRID_E,),
        in_specs=[_espec(16), _espec(16), _espec(1),
                  pl.BlockSpec((16, D), lambda i: (0, 0)), _rspec(),
                  _rspec(), _rspec(), _wspec(), _rspec()],
        out_specs=[_rspec(), _rspec()],
        out_shape=[jax.ShapeDtypeStruct((1, D), _f32)] * 2,
    )(posd, poss, w2, pW1p, pb1, sc1, t1, pW2, pb2)


def _t3_call(posd, poss, Pd, Qs, w2, pW1p, pb1, sc1, t1, pW2, pb2,
             sc2, t2, aW1, ab1):
    return pl.pallas_call(
        _t3_body,
        grid=(GRID_E,),
        in_specs=[_espec(16), _espec(16), _espec(D), _espec(D), _espec(1),
                  pl.BlockSpec((16, D), lambda i: (0, 0)), _rspec(),
                  _rspec(), _rspec(), _wspec(), _rspec(),
                  _rspec(), _rspec(), _wspec(), _rspec()],
        out_specs=[_espec(D), _rspec(), _rspec()],
        out_shape=[jax.ShapeDtypeStruct((EP, D), _f32),
                   jax.ShapeDtypeStruct((1, D), _f32),
                   jax.ShapeDtypeStruct((1, D), _f32)],
    )(posd, poss, Pd, Qs, w2, pW1p, pb1, sc1, t1, pW2, pb2, sc2, t2, aW1, ab1)


def _t4_call(a1p, w2, sc3, t3, aW2, ab2):
    return pl.pallas_call(
        _t4_body,
        grid=(GRID_E,),
        in_specs=[_espec(D), _espec(1), _rspec(), _rspec(), _wspec(), _rspec()],
        out_specs=[_rspec(), _rspec(), _rspec(), _rspec()],
        out_shape=[jax.ShapeDtypeStruct((1, D), _f32)] * 4,
    )(a1p, w2, sc3, t3, aW2, ab2)


def _t5_call(off, nblk, a1p, posd, poss, xvs, w2, pW1p, pb1, sc1, t1, pW2, pb2,
             sc2, t2, sc3, t3, aW2, ab2, sc4, t4, M):
    def _e(cols):
        return pl.BlockSpec((TE, cols), lambda i, _o=off: (i + _o, 0))

    return pl.pallas_call(
        _t5_body,
        grid=(nblk,),
        in_specs=[_e(D), _e(16), _e(16), _e(D), _e(1),
                  pl.BlockSpec((16, D), lambda i: (0, 0)), _rspec(),
                  _rspec(), _rspec(), _wspec(), _rspec(),
                  _rspec(), _rspec(), _rspec(), _rspec(),
                  _wspec(), _rspec(), _rspec(), _rspec(),
                  pl.BlockSpec((1, 1), lambda i: (0, 0))],
        out_specs=[pl.BlockSpec((TE, D), lambda i: (i, 0))] * 2,
        out_shape=[jax.ShapeDtypeStruct((nblk * TE, D), _f32)] * 2,
    )(a1p, posd, poss, xvs, w2, pW1p, pb1, sc1, t1, pW2, pb2, sc2, t2,
      sc3, t3, aW2, ab2, sc4, t4, M)


def _final_body(sa_ref, sb_ref, na_ref, nb_ref, x1_ref, Wup_ref, bup_ref,
                g_ref, b_ref, o_ref):
    sv = sa_ref[...] + sb_ref[...]
    nv = na_ref[...] + nb_ref[...]
    o = jnp.maximum((nv / sv) @ Wup_ref[...] + bup_ref[...], 0.0)
    h = o + x1_ref[...]
    mu = jnp.mean(h, axis=1, keepdims=True)
    var = jnp.mean((h - mu) * (h - mu), axis=1, keepdims=True)
    o_ref[...] = (h - mu) / jnp.sqrt(var + 1e-5) * g_ref[...] + b_ref[...]


def _final_call(sa, sb, na, nb, x1, W_up, b_up, ln_g, ln_b):
    nspec = pl.BlockSpec((NTILE, D), lambda i: (i, 0))
    return pl.pallas_call(
        _final_body,
        grid=(GRID_N,),
        in_specs=[nspec, nspec, nspec, nspec, nspec,
                  _wspec(), _rspec(), _rspec(), _rspec()],
        out_specs=nspec,
        out_shape=jax.ShapeDtypeStruct((N, D), _f32),
    )(sa, sb, na, nb, x1, W_up, b_up, ln_g, ln_b)


def _bn_params(s, ss, cnt, g, b):
    mu = s / cnt
    var = ss / cnt - mu * mu
    sc = g[None, :] / jnp.sqrt(var + 1e-5)
    t = b[None, :] - mu * sc
    return sc, t


def kernel(x, pos, edge_index, W_in, b_in, W_lin, W_src, W_dst, pW1, pb1,
           pg1, pB1, pW2, pb2, pg2, pB2, aW1, ab1, ag1, aB1, aW2, ab2,
           ag2, aB2, W_up, b_up, ln_g, ln_b):
    src = edge_index[0]
    dst = edge_index[1]
    loop = jnp.arange(N, dtype=src.dtype)
    pad = EP - (E + N)
    srcA = jnp.concatenate([src, loop, jnp.zeros((pad,), src.dtype)])
    dstA = jnp.concatenate([dst, loop, jnp.zeros((pad,), src.dtype)])
    w2 = jnp.concatenate([(src != dst).astype(_f32), jnp.ones((N,), _f32),
                          jnp.zeros((pad,), _f32)]).reshape(EP, 1)
    pos16 = jnp.pad(pos, ((0, 0), (0, 13)))
    pW1p = jnp.pad(pW1, ((0, 13), (0, 0)))
    b_in2 = b_in.reshape(1, D)
    pb1r = pb1.reshape(1, D)
    pb2r = pb2.reshape(1, D)
    ab1r = ab1.reshape(1, D)
    ab2r = ab2.reshape(1, D)

    posd, poss = _gather_pos_call(dstA, srcA, pos16)

    x1, xv, Pp, Qp = _node_call(x, W_in, b_in2, W_lin, W_dst, W_src, aW1)

    Pd, Qs, xvs = _gather_main_call(dstA, srcA, Pp, Qp, xv)

    s1, ss1, cntv = _t1_call(posd, poss, w2, pW1p, pb1r)
    cnt = cntv[0, 0]
    sc1, t1 = _bn_params(s1, ss1, cnt, pg1, pB1)

    s2, ss2 = _t2_call(posd, poss, w2, pW1p, pb1r, sc1, t1, pW2, pb2r)
    sc2, t2 = _bn_params(s2, ss2, cnt, pg2, pB2)

    a1p, s3, ss3 = _t3_call(posd, poss, Pd, Qs, w2, pW1p, pb1r, sc1, t1,
                            pW2, pb2r, sc2, t2, aW1, ab1r)
    sc3, t3 = _bn_params(s3, ss3, cnt, ag1, aB1)

    s4, ss4, mxc, mnc = _t4_call(a1p, w2, sc3, t3, aW2, ab2r)
    sc4, t4 = _bn_params(s4, ss4, cnt, ag2, aB2)
    M = jnp.maximum(jnp.maximum(mxc * sc4 + t4, mnc * sc4 + t4), 0.0).max()

    Mr = M.reshape(1, 1)
    zeros_nd = jnp.zeros((N_PAD, D), _f32)
    dstA_a = dstA[:EP_A]
    dstA_b = dstA[EP_A:]

    ex_a, evd_a = _t5_call(0, GRID_A, a1p, posd, poss, xvs, w2, pW1p, pb1r,
                           sc1, t1, pW2, pb2r, sc2, t2, sc3, t3, aW2, ab2r,
                           sc4, t4, Mr)
    s_a, n_a = _scatter_call(dstA_a, ex_a, evd_a, zeros_nd)
    ex_b, evd_b = _t5_call(GRID_A, GRID_B, a1p, posd, poss, xvs, w2, pW1p, pb1r,
                           sc1, t1, pW2, pb2r, sc2, t2, sc3, t3, aW2, ab2r,
                           sc4, t4, Mr)
    s_b, n_b = _scatter_call(dstA_b, ex_b, evd_b, zeros_nd)

    return _final_call(s_a[:N], s_b[:N], n_a[:N], n_b[:N], x1, W_up,
                       b_up.reshape(1, D), ln_g.reshape(1, D),
                       ln_b.reshape(1, D))


# speedup vs baseline: 1.4482x; 1.0447x over previous
import functools
import jax
import jax.numpy as jnp
from jax import lax
from jax.experimental import pallas as pl
from jax.experimental.pallas import tpu as pltpu
from jax.experimental.pallas import tpu_sc as plsc

N = 10000
E = 320000
D = 128
EP = 331776
TE = 4096
GRID_E = EP // TE
NTILE = 1000
GRID_N = N // NTILE

NW = 32
PER_W = EP // NW
CH = 128
NCH = PER_W // CH
GRID_A = 41
GRID_B = GRID_E - GRID_A
EP_A = GRID_A * TE
EP_B = EP - EP_A
N_PAD = 10240
NROW = N_PAD // 16

_f32 = jnp.float32


def _pack2(a, b):
    lo = jax.lax.bitcast_convert_type(a.astype(jnp.bfloat16),
                                      jnp.int16).astype(jnp.int32) & 0xFFFF
    hi = jax.lax.bitcast_convert_type(b.astype(jnp.bfloat16),
                                      jnp.int16).astype(jnp.int32) << 16
    return jax.lax.bitcast_convert_type(lo | hi, jnp.float32)


def _unpack2(w):
    wi = jax.lax.bitcast_convert_type(w, jnp.int32)
    lo = jax.lax.bitcast_convert_type(wi << 16, jnp.float32)
    hi = jax.lax.bitcast_convert_type(wi & jnp.int32(-65536), jnp.float32)
    return jnp.concatenate([lo, hi], axis=1)


def _node_body(x_ref, Win_ref, bin_ref, Wlin_ref, Wd_ref, Ws_ref, aW1_ref,
               x1_ref, Pp_ref, Ts_ref):
    x1 = jnp.maximum(x_ref[...] @ Win_ref[...] + bin_ref[...], 0.0)
    x1_ref[...] = x1
    Pp_ref[...] = x1 @ (Wd_ref[...] @ aW1_ref[...])
    qp = x1 @ (Ws_ref[...] @ aW1_ref[...])
    xv = x1 @ Wlin_ref[...]
    Ts_ref[...] = jnp.concatenate(
        [_pack2(qp[:, :64], qp[:, 64:]), _pack2(xv[:, :64], xv[:, 64:])],
        axis=1)


def _wspec():
    return pl.BlockSpec((D, D), lambda i: (0, 0))


def _rspec():
    return pl.BlockSpec((1, D), lambda i: (0, 0))


def _node_call(x, W_in, b_in, W_lin, W_d, W_s, aW1):
    nspec = pl.BlockSpec((NTILE, D), lambda i: (i, 0))
    outs = [jax.ShapeDtypeStruct((N, D), _f32)] * 3
    return pl.pallas_call(
        _node_body,
        grid=(GRID_N,),
        in_specs=[nspec, _wspec(), _rspec(), _wspec(), _wspec(), _wspec(), _wspec()],
        out_specs=[nspec] * 3,
        out_shape=outs,
    )(x, W_in, b_in, W_lin, W_d, W_s, aW1)


def _gather_pos_call(dstA, srcA, pos16):
    mesh = plsc.VectorSubcoreMesh(core_axis_name="c", subcore_axis_name="s")

    @functools.partial(
        pl.kernel,
        mesh=mesh,
        out_type=[
            jax.ShapeDtypeStruct((EP, 16), _f32),
            jax.ShapeDtypeStruct((EP, 16), _f32),
        ],
        scratch_types=[
            pltpu.VMEM((CH,), jnp.int32),
            pltpu.VMEM((CH,), jnp.int32),
            pltpu.VMEM((CH, 16), _f32),
            pltpu.VMEM((CH, 16), _f32),
            pltpu.SemaphoreType.DMA,
        ],
        compiler_params=pltpu.CompilerParams(use_tc_tiling_on_sc=False),
    )
    def k(dst_hbm, src_hbm, pos_hbm, posd_hbm, poss_hbm,
          idxd, idxs, bpd, bps, sem):
        wid = lax.axis_index("s") * 2 + lax.axis_index("c")
        base = wid * PER_W

        def body(ci, carry):
            off = base + ci * CH
            pltpu.sync_copy(dst_hbm.at[pl.ds(off, CH)], idxd)
            pltpu.sync_copy(src_hbm.at[pl.ds(off, CH)], idxs)
            c4 = pltpu.async_copy(pos_hbm.at[idxd], bpd, sem)
            c5 = pltpu.async_copy(pos_hbm.at[idxs], bps, sem)
            c4.wait(); c5.wait()
            pltpu.sync_copy(bpd, posd_hbm.at[pl.ds(off, CH)])
            pltpu.sync_copy(bps, poss_hbm.at[pl.ds(off, CH)])
            return carry

        lax.fori_loop(0, NCH, body, 0)

    return k(dstA, srcA, pos16)


def _gather_main_call(dstA, srcA, Pp, Ts):
    mesh = plsc.VectorSubcoreMesh(core_axis_name="c", subcore_axis_name="s")

    @functools.partial(
        pl.kernel,
        mesh=mesh,
        out_type=[
            jax.ShapeDtypeStruct((EP, D), _f32),
            jax.ShapeDtypeStruct((EP, D), _f32),
        ],
        scratch_types=[
            pltpu.VMEM((CH,), jnp.int32),
            pltpu.VMEM((CH,), jnp.int32),
            pltpu.VMEM((CH, D), _f32),
            pltpu.VMEM((CH, D), _f32),
            pltpu.SemaphoreType.DMA,
        ],
        compiler_params=pltpu.CompilerParams(use_tc_tiling_on_sc=False),
    )
    def k(dst_hbm, src_hbm, Pp_hbm, Ts_hbm,
          Pd_hbm, Gs_hbm,
          idxd, idxs, bP, bQ, sem):
        wid = lax.axis_index("s") * 2 + lax.axis_index("c")
        base = wid * PER_W

        def body(ci, carry):
            off = base + ci * CH
            pltpu.sync_copy(dst_hbm.at[pl.ds(off, CH)], idxd)
            pltpu.sync_copy(src_hbm.at[pl.ds(off, CH)], idxs)
            c1 = pltpu.async_copy(Pp_hbm.at[idxd], bP, sem)
            c2 = pltpu.async_copy(Ts_hbm.at[idxs], bQ, sem)
            c1.wait(); c2.wait()
            pltpu.sync_copy(bP, Pd_hbm.at[pl.ds(off, CH)])
            pltpu.sync_copy(bQ, Gs_hbm.at[pl.ds(off, CH)])
            return carry

        lax.fori_loop(0, NCH, body, 0)

    return k(dstA, srcA, Pp, Ts)


def _scatter_call(dstA, ex, evd, zeros_nd):
    ep = ex.shape[0]
    per_t = ep // 16
    nch = per_t // CH
    mesh = plsc.VectorSubcoreMesh(core_axis_name="c", subcore_axis_name="s")

    @functools.partial(
        pl.kernel,
        mesh=mesh,
        out_type=[
            jax.ShapeDtypeStruct((N_PAD, D), _f32),
            jax.ShapeDtypeStruct((N_PAD, D), _f32),
        ],
        scratch_types=[
            pltpu.VMEM((CH,), jnp.int32),
            pltpu.VMEM((CH, D), _f32),
            pltpu.VMEM_SHARED((N_PAD, D), _f32),
        ],
    )
    def k(dst_hbm, ex_hbm, evd_hbm, z_hbm, s_hbm, num_hbm, idx, rows, acc):
        c = lax.axis_index("c")
        sid = lax.axis_index("s")
        pltpu.sync_copy(z_hbm.at[pl.ds(sid * NROW, NROW)],
                        acc.at[pl.ds(sid * NROW, NROW)])
        plsc.subcore_barrier()

        def run(data_hbm):
            def body(ci, carry):
                off = sid * per_t + ci * CH
                pltpu.sync_copy(dst_hbm.at[pl.ds(off, CH)], idx)
                pltpu.sync_copy(data_hbm.at[pl.ds(off, CH)], rows)
                pltpu.sync_copy(rows, acc.at[idx], add=True)
                return carry
            lax.fori_loop(0, nch, body, 0)

        @pl.when(c == 0)
        def _():
            run(ex_hbm)

        @pl.when(c == 1)
        def _():
            run(evd_hbm)

        plsc.subcore_barrier()

        @pl.when(c == 0)
        def _():
            pltpu.sync_copy(acc.at[pl.ds(sid * NROW, NROW)],
                            s_hbm.at[pl.ds(sid * NROW, NROW)])

        @pl.when(c == 1)
        def _():
            pltpu.sync_copy(acc.at[pl.ds(sid * NROW, NROW)],
                            num_hbm.at[pl.ds(sid * NROW, NROW)])

    return k(dstA, ex, evd, zeros_nd)


def _bmm(a, b):
    return jax.lax.dot(a.astype(jnp.bfloat16), b.astype(jnp.bfloat16),
                       preferred_element_type=jnp.float32)


def _espec(cols, colblk=0):
    return pl.BlockSpec((TE, cols), lambda i, _c=colblk: (i, _c))


def _acc_init(ref):
    @pl.when(pl.program_id(0) == 0)
    def _():
        ref[...] = jnp.zeros_like(ref)


def _t1_body(posd_ref, poss_ref, w_ref, pW1_ref, pb1_ref,
             s_ref, ss_ref, cnt_ref):
    rel = posd_ref[...] - poss_ref[...]
    h1 = rel @ pW1_ref[...] + pb1_ref[...]
    w = w_ref[...]
    wh = h1 * w
    _acc_init(s_ref); _acc_init(ss_ref); _acc_init(cnt_ref)
    s_ref[...] += jnp.sum(wh, axis=0)[None, :]
    ss_ref[...] += jnp.sum(wh * h1, axis=0)[None, :]
    cnt_ref[...] += jnp.full((1, D), jnp.sum(w), _f32)


def _t2_body(posd_ref, poss_ref, w_ref, pW1_ref, pb1_ref, sc1_ref, t1_ref,
             pW2_ref, pb2_ref, s_ref, ss_ref):
    rel = posd_ref[...] - poss_ref[...]
    h1 = rel @ pW1_ref[...] + pb1_ref[...]
    r = jnp.maximum(h1 * sc1_ref[...] + t1_ref[...], 0.0)
    h2 = _bmm(r, pW2_ref[...]) + pb2_ref[...]
    w = w_ref[...]
    wh = h2 * w
    _acc_init(s_ref); _acc_init(ss_ref)
    s_ref[...] += jnp.sum(wh, axis=0)[None, :]
    ss_ref[...] += jnp.sum(wh * h2, axis=0)[None, :]


def _delta(posd_ref, poss_ref, pW1_ref, pb1_ref, sc1_ref, t1_ref,
           pW2_ref, pb2_ref, sc2_ref, t2_ref):
    rel = posd_ref[...] - poss_ref[...]
    h1 = rel @ pW1_ref[...] + pb1_ref[...]
    r = jnp.maximum(h1 * sc1_ref[...] + t1_ref[...], 0.0)
    h2 = _bmm(r, pW2_ref[...]) + pb2_ref[...]
    return jnp.maximum(h2 * sc2_ref[...] + t2_ref[...], 0.0)


def _t3_body(posd_ref, poss_ref, Pd_ref, Qs_ref, w_ref,
             pW1_ref, pb1_ref, sc1_ref, t1_ref, pW2_ref, pb2_ref,
             sc2_ref, t2_ref, aW1_ref, ab1_ref,
             a1p_ref, s_ref, ss_ref):
    delta = _delta(posd_ref, poss_ref, pW1_ref, pb1_ref, sc1_ref, t1_ref,
                   pW2_ref, pb2_ref, sc2_ref, t2_ref)
    a1p = (Pd_ref[...] - _unpack2(Qs_ref[...][:, :64])
           + _bmm(delta, aW1_ref[...]) + ab1_ref[...])
    a1p_ref[...] = a1p
    w = w_ref[...]
    wh = a1p * w
    _acc_init(s_ref); _acc_init(ss_ref)
    s_ref[...] += jnp.sum(wh, axis=0)[None, :]
    ss_ref[...] += jnp.sum(wh * a1p, axis=0)[None, :]


def _t4_body(a1p_ref, w_ref, sc3_ref, t3_ref, aW2_ref, ab2_ref,
             s_ref, ss_ref, mx_ref, mn_ref):
    ra = jnp.maximum(a1p_ref[...] * sc3_ref[...] + t3_ref[...], 0.0)
    a2p = _bmm(ra, aW2_ref[...]) + ab2_ref[...]
    w = w_ref[...]
    wh = a2p * w
    _acc_init(s_ref); _acc_init(ss_ref)
    s_ref[...] += jnp.sum(wh, axis=0)[None, :]
    ss_ref[...] += jnp.sum(wh * a2p, axis=0)[None, :]
    pmx = jnp.max(a2p, axis=0)[None, :]
    pmn = jnp.min(a2p, axis=0)[None, :]

    @pl.when(pl.program_id(0) == 0)
    def _():
        mx_ref[...] = pmx
        mn_ref[...] = pmn

    @pl.when(pl.program_id(0) > 0)
    def _():
        mx_ref[...] = jnp.maximum(mx_ref[...], pmx)
        mn_ref[...] = jnp.minimum(mn_ref[...], pmn)


def _t5_body(a1p_ref, posd_ref, poss_ref, xvs_ref, w_ref,
             pW1_ref, pb1_ref, sc1_ref, t1_ref, pW2_ref, pb2_ref,
             sc2_ref, t2_ref, sc3_ref, t3_ref, aW2_ref, ab2_ref,
             sc4_ref, t4_ref, M_ref,
             ex_ref, evd_ref):
    delta = _delta(posd_ref, poss_ref, pW1_ref, pb1_ref, sc1_ref, t1_ref,
                   pW2_ref, pb2_ref, sc2_ref, t2_ref)
    ra = jnp.maximum(a1p_ref[...] * sc3_ref[...] + t3_ref[...], 0.0)
    a2p = _bmm(ra, aW2_ref[...]) + ab2_ref[...]
    af = jnp.maximum(a2p * sc4_ref[...] + t4_ref[...], 0.0)
    ex = w_ref[...] * jnp.exp(af - M_ref[...])
    ex_ref[...] = ex
    evd_ref[...] = ex * (_unpack2(xvs_ref[...][:, 64:]) + delta)


def _t1_call(posd, poss, w2, pW1p, pb1):
    return pl.pallas_call(
        _t1_body,
        grid=(GRID_E,),
        in_specs=[_espec(16), _espec(16), _espec(1),
                  pl.BlockSpec((16, D), lambda i: (0, 0)), _rspec()],
        out_specs=[_rspec(), _rspec(), _rspec()],
        out_shape=[jax.ShapeDtypeStruct((1, D), _f32)] * 3,
    )(posd, poss, w2, pW1p, pb1)


def _t2_call(posd, poss, w2, pW1p, pb1, sc1, t1, pW2, pb2):
    return pl.pallas_call(
        _t2_body,
        grid=(GRID_E,),
        in_specs=[_espec(16), _espec(16), _espec(1),
                  pl.BlockSpec((16, D), lambda i: (0, 0)), _rspec(),
                  _rspec(), _rspec(), _wspec(), _rspec()],
        out_specs=[_rspec(), _rspec()],
        out_shape=[jax.ShapeDtypeStruct((1, D), _f32)] * 2,
    )(posd, poss, w2, pW1p, pb1, sc1, t1, pW2, pb2)


def _t3_call(posd, poss, Pd, Qs, w2, pW1p, pb1, sc1, t1, pW2, pb2,
             sc2, t2, aW1, ab1):
    return pl.pallas_call(
        _t3_body,
        grid=(GRID_E,),
        in_specs=[_espec(16), _espec(16), _espec(D), _espec(D), _espec(1),
                  pl.BlockSpec((16, D), lambda i: (0, 0)), _rspec(),
                  _rspec(), _rspec(), _wspec(), _rspec(),
                  _rspec(), _rspec(), _wspec(), _rspec()],
        out_specs=[_espec(D), _rspec(), _rspec()],
        out_shape=[jax.ShapeDtypeStruct((EP, D), _f32),
                   jax.ShapeDtypeStruct((1, D), _f32),
                   jax.ShapeDtypeStruct((1, D), _f32)],
    )(posd, poss, Pd, Qs, w2, pW1p, pb1, sc1, t1, pW2, pb2, sc2, t2, aW1, ab1)


def _t4_call(a1p, w2, sc3, t3, aW2, ab2):
    return pl.pallas_call(
        _t4_body,
        grid=(GRID_E,),
        in_specs=[_espec(D), _espec(1), _rspec(), _rspec(), _wspec(), _rspec()],
        out_specs=[_rspec(), _rspec(), _rspec(), _rspec()],
        out_shape=[jax.ShapeDtypeStruct((1, D), _f32)] * 4,
    )(a1p, w2, sc3, t3, aW2, ab2)


def _t5_call(off, nblk, a1p, posd, poss, xvs, w2, pW1p, pb1, sc1, t1, pW2, pb2,
             sc2, t2, sc3, t3, aW2, ab2, sc4, t4, M):
    def _e(cols, colblk=0):
        return pl.BlockSpec((TE, cols),
                            lambda i, _o=off, _c=colblk: (i + _o, _c))

    return pl.pallas_call(
        _t5_body,
        grid=(nblk,),
        in_specs=[_e(D), _e(16), _e(16), _e(D), _e(1),
                  pl.BlockSpec((16, D), lambda i: (0, 0)), _rspec(),
                  _rspec(), _rspec(), _wspec(), _rspec(),
                  _rspec(), _rspec(), _rspec(), _rspec(),
                  _wspec(), _rspec(), _rspec(), _rspec(),
                  pl.BlockSpec((1, 1), lambda i: (0, 0))],
        out_specs=[pl.BlockSpec((TE, D), lambda i: (i, 0))] * 2,
        out_shape=[jax.ShapeDtypeStruct((nblk * TE, D), _f32)] * 2,
    )(a1p, posd, poss, xvs, w2, pW1p, pb1, sc1, t1, pW2, pb2, sc2, t2,
      sc3, t3, aW2, ab2, sc4, t4, M)


def _final_body(sa_ref, sb_ref, na_ref, nb_ref, x1_ref, Wup_ref, bup_ref,
                g_ref, b_ref, o_ref):
    sv = sa_ref[...] + sb_ref[...]
    nv = na_ref[...] + nb_ref[...]
    o = jnp.maximum((nv / sv) @ Wup_ref[...] + bup_ref[...], 0.0)
    h = o + x1_ref[...]
    mu = jnp.mean(h, axis=1, keepdims=True)
    var = jnp.mean((h - mu) * (h - mu), axis=1, keepdims=True)
    o_ref[...] = (h - mu) / jnp.sqrt(var + 1e-5) * g_ref[...] + b_ref[...]


def _final_call(sa, sb, na, nb, x1, W_up, b_up, ln_g, ln_b):
    nspec = pl.BlockSpec((NTILE, D), lambda i: (i, 0))
    return pl.pallas_call(
        _final_body,
        grid=(GRID_N,),
        in_specs=[nspec, nspec, nspec, nspec, nspec,
                  _wspec(), _rspec(), _rspec(), _rspec()],
        out_specs=nspec,
        out_shape=jax.ShapeDtypeStruct((N, D), _f32),
    )(sa, sb, na, nb, x1, W_up, b_up, ln_g, ln_b)


def _bn_params(s, ss, cnt, g, b):
    mu = s / cnt
    var = ss / cnt - mu * mu
    sc = g[None, :] / jnp.sqrt(var + 1e-5)
    t = b[None, :] - mu * sc
    return sc, t


def kernel(x, pos, edge_index, W_in, b_in, W_lin, W_src, W_dst, pW1, pb1,
           pg1, pB1, pW2, pb2, pg2, pB2, aW1, ab1, ag1, aB1, aW2, ab2,
           ag2, aB2, W_up, b_up, ln_g, ln_b):
    src = edge_index[0]
    dst = edge_index[1]
    loop = jnp.arange(N, dtype=src.dtype)
    pad = EP - (E + N)
    srcA = jnp.concatenate([src, loop, jnp.zeros((pad,), src.dtype)])
    dstA = jnp.concatenate([dst, loop, jnp.zeros((pad,), src.dtype)])
    w2 = jnp.concatenate([(src != dst).astype(_f32), jnp.ones((N,), _f32),
                          jnp.zeros((pad,), _f32)]).reshape(EP, 1)
    pos16 = jnp.pad(pos, ((0, 0), (0, 13)))
    pW1p = jnp.pad(pW1, ((0, 13), (0, 0)))
    b_in2 = b_in.reshape(1, D)
    pb1r = pb1.reshape(1, D)
    pb2r = pb2.reshape(1, D)
    ab1r = ab1.reshape(1, D)
    ab2r = ab2.reshape(1, D)

    posd, poss = _gather_pos_call(dstA, srcA, pos16)

    x1, Pp, Ts = _node_call(x, W_in, b_in2, W_lin, W_dst, W_src, aW1)

    Pd, Gs = _gather_main_call(dstA, srcA, Pp, Ts)

    s1, ss1, cntv = _t1_call(posd, poss, w2, pW1p, pb1r)
    cnt = cntv[0, 0]
    sc1, t1 = _bn_params(s1, ss1, cnt, pg1, pB1)

    s2, ss2 = _t2_call(posd, poss, w2, pW1p, pb1r, sc1, t1, pW2, pb2r)
    sc2, t2 = _bn_params(s2, ss2, cnt, pg2, pB2)

    a1p, s3, ss3 = _t3_call(posd, poss, Pd, Gs, w2, pW1p, pb1r, sc1, t1,
                            pW2, pb2r, sc2, t2, aW1, ab1r)
    sc3, t3 = _bn_params(s3, ss3, cnt, ag1, aB1)

    s4, ss4, mxc, mnc = _t4_call(a1p, w2, sc3, t3, aW2, ab2r)
    sc4, t4 = _bn_params(s4, ss4, cnt, ag2, aB2)
    M = jnp.maximum(jnp.maximum(mxc * sc4 + t4, mnc * sc4 + t4), 0.0).max()

    Mr = M.reshape(1, 1)
    zeros_nd = jnp.zeros((N_PAD, D), _f32)
    dstA_a = dstA[:EP_A]
    dstA_b = dstA[EP_A:]

    ex_a, evd_a = _t5_call(0, GRID_A, a1p, posd, poss, Gs, w2, pW1p, pb1r,
                           sc1, t1, pW2, pb2r, sc2, t2, sc3, t3, aW2, ab2r,
                           sc4, t4, Mr)
    s_a, n_a = _scatter_call(dstA_a, ex_a, evd_a, zeros_nd)
    ex_b, evd_b = _t5_call(GRID_A, GRID_B, a1p, posd, poss, Gs, w2, pW1p, pb1r,
                           sc1, t1, pW2, pb2r, sc2, t2, sc3, t3, aW2, ab2r,
                           sc4, t4, Mr)
    s_b, n_b = _scatter_call(dstA_b, ex_b, evd_b, zeros_nd)

    return _final_call(s_a[:N], s_b[:N], n_a[:N], n_b[:N], x1, W_up,
                       b_up.reshape(1, D), ln_g.reshape(1, D),
                       ln_b.reshape(1, D))
